# Initial kernel scaffold; baseline (speedup 1.0000x reference)
import functools

import jax
import jax.numpy as jnp
from jax import lax
from jax.experimental import pallas as pl
from jax.experimental.pallas import tpu as pltpu
from jax.experimental.pallas import tpu_sc as plsc

N = 50000
E = 800000
G = 512
BLK = 2000
NB = N // BLK
D = 32

NSC = 2
NTILE = 16
NW = NSC * NTILE
CH = 128
K = 14
TPT = 196
EPAD = NW * TPT * CH
NPAD = 50016
RPT = NPAD // NTILE


def _sc_agg_body(p_hbm, src_hbm, dst_hbm, zeros_hbm, out_hbm,
                 src_v, dst_v, rows_v, acc_sh, sem0, sem1):
    c = lax.axis_index("c")
    s = lax.axis_index("s")
    wid = c * NTILE + s
    sems = (sem0, sem1)

    pltpu.sync_copy(zeros_hbm, acc_sh.at[pl.ds(s * RPT, RPT)])
    plsc.subcore_barrier()

    base = wid * TPT

    def outer(b, carry):
        row0 = base + b * K
        pltpu.sync_copy(src_hbm.at[pl.ds(row0, K)], src_v)
        pltpu.sync_copy(dst_hbm.at[pl.ds(row0, K)], dst_v)
        descs = [None, None]
        descs[0] = pltpu.async_copy(p_hbm.at[src_v.at[0]], rows_v.at[0],
                                    sems[0])
        for j in range(K):
            if j + 1 < K:
                nb = (j + 1) % 2
                descs[nb] = pltpu.async_copy(
                    p_hbm.at[src_v.at[j + 1]], rows_v.at[nb], sems[nb])
            descs[j % 2].wait()
            pltpu.sync_copy(rows_v.at[j % 2], acc_sh.at[dst_v.at[j]],
                            add=True)
        return carry

    lax.fori_loop(0, TPT // K, outer, 0)
    plsc.subcore_barrier()

    pltpu.sync_copy(acc_sh.at[pl.ds(s * RPT, RPT)],
                    out_hbm.at[c, pl.ds(s * RPT, RPT)])


def _sc_aggregate(p, src2d, dst2d, zeros_init):
    mesh = plsc.VectorSubcoreMesh(core_axis_name="c", subcore_axis_name="s")
    f = pl.kernel(
        _sc_agg_body,
        out_type=jax.ShapeDtypeStruct((NSC, NPAD, D), jnp.float32),
        mesh=mesh,
        scratch_types=[
            pltpu.VMEM((K, CH), jnp.int32),
            pltpu.VMEM((K, CH), jnp.int32),
            pltpu.VMEM((2, CH, D), jnp.float32),
            pltpu.VMEM_SHARED((NPAD, D), jnp.float32),
            pltpu.SemaphoreType.DMA,
            pltpu.SemaphoreType.DMA,
        ],
        name="gin_edge_agg",
    )
    return f(p, src2d, dst2d, zeros_init)


def _proj_first_body(x_ref, w_ref, o_ref):
    o_ref[...] = jnp.dot(x_ref[...], w_ref[...],
                         preferred_element_type=jnp.float32)


def _proj_first(x_pad, w_pad):
    return pl.pallas_call(
        _proj_first_body,
        grid=(NB,),
        in_specs=[
            pl.BlockSpec((BLK, 128), lambda i: (i, 0)),
            pl.BlockSpec((128, D), lambda i: (0, 0)),
        ],
        out_specs=pl.BlockSpec((BLK, D), lambda i: (i, 0)),
        out_shape=jax.ShapeDtypeStruct((N, D), jnp.float32),
    )(x_pad, w_pad)


def _proj_bn_body(h_ref, st_ref, w_ref, o_ref):
    mu = st_ref[0:1, :] * (1.0 / N)
    ex2 = st_ref[1:2, :] * (1.0 / N)
    inv = lax.rsqrt(ex2 - mu * mu + 1e-5)
    hn = (h_ref[...] - mu) * inv
    o_ref[...] = jnp.dot(hn, w_ref[...], preferred_element_type=jnp.float32)


def _proj_bn(h, stats, w):
    return pl.pallas_call(
        _proj_bn_body,
        grid=(NB,),
        in_specs=[
            pl.BlockSpec((BLK, D), lambda i: (i, 0)),
            pl.BlockSpec((8, D), lambda i: (0, 0)),
            pl.BlockSpec((D, D), lambda i: (0, 0)),
        ],
        out_specs=pl.BlockSpec((BLK, D), lambda i: (i, 0)),
        out_shape=jax.ShapeDtypeStruct((N, D), jnp.float32),
    )(h, stats, w)


def _layer_body(act, pool, p_ref, a0_ref, a1_ref, b1_ref, w2_ref, b2_ref,
                batch_ref, h_ref, st_ref, pooled_ref):
    i = pl.program_id(0)
    z = jnp.maximum(p_ref[...] + a0_ref[...] + a1_ref[...] + b1_ref[0:1, :],
                    0.0)
    h = jnp.dot(z, w2_ref[...], preferred_element_type=jnp.float32)
    h = h + b2_ref[0:1, :]
    if act == "elu":
        h = jnp.where(h > 0, h, jnp.exp(jnp.minimum(h, 0.0)) - 1.0)
    else:
        h = jnp.maximum(h, 0.0)
    h_ref[...] = h

    s1 = jnp.sum(h, axis=0, keepdims=True)
    s2 = jnp.sum(h * h, axis=0, keepdims=True)
    rows = lax.broadcasted_iota(jnp.int32, (8, D), 0)
    upd = jnp.where(rows == 0, s1, 0.0) + jnp.where(rows == 1, s2, 0.0)

    @pl.when(i == 0)
    def _():
        st_ref[...] = jnp.zeros_like(st_ref)

    st_ref[...] += upd

    if pool:
        bb = batch_ref[0, 0, :].reshape(1, BLK)
        gio = lax.broadcasted_iota(jnp.int32, (G, BLK), 0)
        oht = (gio == bb).astype(jnp.float32)
        hx = jnp.concatenate(
            [h, jnp.ones((BLK, 8), jnp.float32)], axis=1)
        pu = jnp.dot(oht, hx, preferred_element_type=jnp.float32)

        @pl.when(i == 0)
        def _():
            pooled_ref[...] = jnp.zeros_like(pooled_ref)

        pooled_ref[...] += pu


def _layer(act, pool, p, a0, a1, b1r, w2, b2r, batch3d):
    body = functools.partial(_layer_body, act, pool)
    out_shape = [
        jax.ShapeDtypeStruct((N, D), jnp.float32),
        jax.ShapeDtypeStruct((8, D), jnp.float32),
        jax.ShapeDtypeStruct((G, D + 8), jnp.float32),
    ]
    out_specs = [
        pl.BlockSpec((BLK, D), lambda i: (i, 0)),
        pl.BlockSpec((8, D), lambda i: (0, 0)),
        pl.BlockSpec((G, D + 8), lambda i: (0, 0)),
    ]
    return pl.pallas_call(
        body,
        grid=(NB,),
        in_specs=[
            pl.BlockSpec((BLK, D), lambda i: (i, 0)),
            pl.BlockSpec((BLK, D), lambda i: (i, 0)),
            pl.BlockSpec((BLK, D), lambda i: (i, 0)),
            pl.BlockSpec((8, D), lambda i: (0, 0)),
            pl.BlockSpec((D, D), lambda i: (0, 0)),
            pl.BlockSpec((8, D), lambda i: (0, 0)),
            pl.BlockSpec((1, 1, BLK), lambda i: (i, 0, 0)),
        ],
        out_specs=out_specs,
        out_shape=out_shape,
    )(p, a0, a1, b1r, w2, b2r, batch3d)


def _head_body(pooled_ref, st_ref, target_ref, g_ref, bb_ref, n2w_ref,
               n2b_ref, n31w_ref, n31b_ref, n32w_ref, n32b_ref, n4t_ref,
               n4b_ref, n4bias_ref, n5w_ref, n5b_ref, out_ref, xg_ref):
    mu = st_ref[0:1, :] * (1.0 / N)
    ex2 = st_ref[1:2, :] * (1.0 / N)
    inv = lax.rsqrt(ex2 - mu * mu + 1e-5)
    praw = pooled_ref[:, 0:D]
    cnt = pooled_ref[:, D:D + 1]
    pooled = (praw - cnt * mu) * inv

    xg = jnp.dot(pooled, n2w_ref[...], preferred_element_type=jnp.float32)
    xg = jnp.maximum(xg + n2b_ref[0:1, :], 0.0)
    xg_ref[...] = xg

    t = target_ref[...]
    tm = jnp.mean(t, axis=0, keepdims=True)
    tv = jnp.mean((t - tm) ** 2, axis=0, keepdims=True)
    tn = (t - tm) / jnp.sqrt(tv + 1e-5)
    tn = tn * g_ref[0:1, :] + bb_ref[0:1, :]

    c = jnp.dot(tn, n31w_ref[...], preferred_element_type=jnp.float32)
    c = c + n31b_ref[0:1, :]
    c = jnp.dot(c, n32w_ref[...], preferred_element_type=jnp.float32)
    c = c + n32b_ref[0:1, :]
    c = c - jnp.max(c, axis=1, keepdims=True)
    ec = jnp.exp(c)
    sm = ec / jnp.sum(ec, axis=1, keepdims=True)

    xc = (jnp.dot(xg, n4t_ref[...], preferred_element_type=jnp.float32)
          + jnp.dot(sm, n4b_ref[...], preferred_element_type=jnp.float32)
          + n4bias_ref[0:1, :])
    xc = jnp.maximum(xc, 0.0)

    o = jnp.dot(xc, n5w_ref[...], preferred_element_type=jnp.float32)
    o = o[:, 0:1] + n5b_ref[0:1, 0:1]
    out_ref[...] = jax.nn.sigmoid(o)


def _head(pooled, st3, target, g8, b8, n2w, n2b8, n31w, n31b8, n32w, n32b8,
          n4t, n4b, n4bias8, n5wp, n5b8):
    return pl.pallas_call(
        _head_body,
        out_shape=[
            jax.ShapeDtypeStruct((G, 1), jnp.float32),
            jax.ShapeDtypeStruct((G, 128), jnp.float32),
        ],
    )(pooled, st3, target, g8, b8, n2w, n2b8, n31w, n31b8, n32w, n32b8,
      n4t, n4b, n4bias8, n5wp, n5b8)


def _row8(v):
    return jnp.broadcast_to(v.reshape(1, -1), (8, v.shape[0]))


def kernel(x, edge_index, batch, target, n11_W1, n11_b1, n11_W2, n11_b2,
           n12_W1, n12_b1, n12_W2, n12_b2, n13_W1, n13_b1, n13_W2, n13_b2,
           n2_W, n2_b, n31_W, n31_b, n32_W, n32_b, n4_W, n4_b, n5_W, n5_b,
           bn1_g, bn1_b):
    src = edge_index[0]
    dst = edge_index[1]
    src2d = jnp.concatenate(
        [src, jnp.zeros((EPAD - E,), jnp.int32)]).reshape(EPAD // CH, CH)
    dst2d = jnp.concatenate(
        [dst, jnp.full((EPAD - E,), N, jnp.int32)]).reshape(EPAD // CH, CH)
    zeros_init = jnp.zeros((RPT, D), jnp.float32)
    x_pad = jnp.pad(x, ((0, 0), (0, 128 - x.shape[1])))
    w1_pad = jnp.pad(n11_W1, ((0, 128 - n11_W1.shape[0]), (0, 0)))
    batch3d = batch.reshape(NB, 1, BLK)

    b11r, b12r = _row8(n11_b1), _row8(n11_b2)
    b21r, b22r = _row8(n12_b1), _row8(n12_b2)
    b31r, b32r = _row8(n13_b1), _row8(n13_b2)

    p1 = _proj_first(x_pad, w1_pad)
    agg1 = _sc_aggregate(p1, src2d, dst2d, zeros_init)
    h1, st1, _ = _layer("elu", False, p1, agg1[0, :N], agg1[1, :N],
                        b11r, n11_W2, b12r, batch3d)

    p2 = _proj_bn(h1, st1, n12_W1)
    agg2 = _sc_aggregate(p2, src2d, dst2d, zeros_init)
    h2, st2, _ = _layer("relu", False, p2, agg2[0, :N], agg2[1, :N],
                        b21r, n12_W2, b22r, batch3d)

    p3 = _proj_bn(h2, st2, n13_W1)
    agg3 = _sc_aggregate(p3, src2d, dst2d, zeros_init)
    _, st3, pooled = _layer("relu", True, p3, agg3[0, :N], agg3[1, :N],
                            b31r, n13_W2, b32r, batch3d)

    n4t, n4b = n4_W[:128], n4_W[128:]
    n5wp = jnp.pad(n5_W, ((0, 0), (0, 7)))
    out, xg = _head(pooled, st3, target, _row8(bn1_g), _row8(bn1_b),
                    n2_W, _row8(n2_b), n31_W, _row8(n31_b), n32_W,
                    _row8(n32_b), n4t, n4b, _row8(n4_b), n5wp, _row8(n5_b))
    return (out, xg)


# speedup vs baseline: 5.1010x; 5.1010x over previous
import functools

import jax
import jax.numpy as jnp
from jax import lax
from jax.experimental import pallas as pl
from jax.experimental.pallas import tpu as pltpu
from jax.experimental.pallas import tpu_sc as plsc

N = 50000
E = 800000
G = 512
BLK = 2000
NB = N // BLK
D = 32
F1 = 80
FH = 32
FT = 16

NSC = 2
NTILE = 16
NW = NSC * NTILE
CH = 128
K = 8
NPAD = 50048
RPT = NPAD // NTILE

TPT_W = 200
EPAD_W = NW * TPT_W * CH
TPT_F = 392
EPAD_F = NTILE * TPT_F * CH

BF = jnp.bfloat16


def _bdot(a, b):
    return jnp.dot(a.astype(BF), b.astype(BF),
                   preferred_element_type=jnp.float32)


def _edge_loop(table_hbm, src_hbm, dst_hbm, acc_sh, src_v, dst_v, rows_v,
               sems, base, tpt):
    def outer(b, carry):
        row0 = base + b * K
        pltpu.sync_copy(src_hbm.at[pl.ds(row0, K)], src_v)
        pltpu.sync_copy(dst_hbm.at[pl.ds(row0, K)], dst_v)
        descs = [None, None]
        descs[0] = pltpu.async_copy(table_hbm.at[src_v.at[0]], rows_v.at[0],
                                    sems[0])
        for j in range(K):
            if j + 1 < K:
                nb = (j + 1) % 2
                descs[nb] = pltpu.async_copy(
                    table_hbm.at[src_v.at[j + 1]], rows_v.at[nb], sems[nb])
            descs[j % 2].wait()
            pltpu.sync_copy(rows_v.at[j % 2], acc_sh.at[dst_v.at[j]],
                            add=True)
        return carry

    lax.fori_loop(0, tpt // K, outer, 0)


def _sc_agg_edges_body(p_hbm, src_hbm, dst_hbm, zeros_hbm, out_hbm,
                       src_v, dst_v, rows_v, acc_sh, sem0, sem1):
    c = lax.axis_index("c")
    s = lax.axis_index("s")
    wid = c * NTILE + s
    pltpu.sync_copy(zeros_hbm, acc_sh.at[pl.ds(s * RPT, RPT)])
    plsc.subcore_barrier()
    _edge_loop(p_hbm, src_hbm, dst_hbm, acc_sh, src_v, dst_v, rows_v,
               (sem0, sem1), wid * TPT_W, TPT_W)
    plsc.subcore_barrier()
    pltpu.sync_copy(acc_sh.at[pl.ds(s * RPT, RPT)],
                    out_hbm.at[c, pl.ds(s * RPT, RPT)])


def _sc_aggregate(p, src2d, dst2d, zeros_init, width=D):
    mesh = plsc.VectorSubcoreMesh(core_axis_name="c", subcore_axis_name="s")
    f = pl.kernel(
        _sc_agg_edges_body,
        out_type=jax.ShapeDtypeStruct((NSC, NPAD, width), jnp.float32),
        mesh=mesh,
        scratch_types=[
            pltpu.VMEM((K, CH), jnp.int32),
            pltpu.VMEM((K, CH), jnp.int32),
            pltpu.VMEM((2, CH, width), jnp.float32),
            pltpu.VMEM_SHARED((NPAD, width), jnp.float32),
            pltpu.SemaphoreType.DMA,
            pltpu.SemaphoreType.DMA,
        ],
        compiler_params=pltpu.CompilerParams(use_tc_tiling_on_sc=False),
        name="gin_edge_agg",
    )
    return f(p, src2d, dst2d, zeros_init)


def _sc_agg_feat_body(xl_hbm, xr_hbm, src_hbm, dst_hbm, zeros_hbm, out_hbm,
                      src_v, dst_v, rows_v, acc_sh, sem0, sem1):
    c = lax.axis_index("c")
    s = lax.axis_index("s")
    pltpu.sync_copy(zeros_hbm, acc_sh.at[pl.ds(s * RPT, RPT)])
    plsc.subcore_barrier()

    @pl.when(c == 0)
    def _():
        _edge_loop(xl_hbm, src_hbm, dst_hbm, acc_sh, src_v, dst_v, rows_v,
                   (sem0, sem1), s * TPT_F, TPT_F)

    @pl.when(c == 1)
    def _():
        _edge_loop(xr_hbm, src_hbm, dst_hbm, acc_sh, src_v, dst_v, rows_v,
                   (sem0, sem1), s * TPT_F, TPT_F)

    plsc.subcore_barrier()
    pltpu.sync_copy(acc_sh.at[pl.ds(s * RPT, RPT)],
                    out_hbm.at[c, pl.ds(s * RPT, RPT)])


def _sc_aggregate_x(xl, xr, src2d, dst2d, zeros_init):
    mesh = plsc.VectorSubcoreMesh(core_axis_name="c", subcore_axis_name="s")
    f = pl.kernel(
        _sc_agg_feat_body,
        out_type=jax.ShapeDtypeStruct((NSC, NPAD, FH), jnp.float32),
        mesh=mesh,
        scratch_types=[
            pltpu.VMEM((K, CH), jnp.int32),
            pltpu.VMEM((K, CH), jnp.int32),
            pltpu.VMEM((2, CH, FH), jnp.float32),
            pltpu.VMEM_SHARED((NPAD, FH), jnp.float32),
            pltpu.SemaphoreType.DMA,
            pltpu.SemaphoreType.DMA,
        ],
        compiler_params=pltpu.CompilerParams(use_tc_tiling_on_sc=False),
        name="gin_x_agg",
    )
    return f(xl, xr, src2d, dst2d, zeros_init)


def _stats_update(st_ref, h, i):
    s1 = jnp.sum(h, axis=0, keepdims=True)
    s2 = jnp.sum(h * h, axis=0, keepdims=True)
    rows = lax.broadcasted_iota(jnp.int32, (8, D), 0)
    upd = jnp.where(rows == 0, s1, 0.0) + jnp.where(rows == 1, s2, 0.0)

    @pl.when(i == 0)
    def _():
        st_ref[...] = jnp.zeros_like(st_ref)

    st_ref[...] += upd


def _layer1_body(x_ref, af_ref, e0_ref, e1_ref, b1_ref, w1_ref, w2_ref,
                 b2_ref, h_ref, st_ref):
    i = pl.program_id(0)
    a = jnp.concatenate([af_ref[...], e0_ref[...] + e1_ref[...]], axis=1)
    z = x_ref[...] + a
    z = jnp.concatenate([z, jnp.zeros((BLK, 128 - F1), jnp.float32)],
                        axis=1)
    zr = jnp.maximum(_bdot(z, w1_ref[...]) + b1_ref[0:1, :], 0.0)
    h = _bdot(zr, w2_ref[...]) + b2_ref[0:1, :]
    h = jnp.where(h > 0, h, jnp.exp(jnp.minimum(h, 0.0)) - 1.0)
    h_ref[...] = h
    _stats_update(st_ref, h, i)


def _layer1(x_pad, aggf, e0, e1, b1r, w1_pad, w2, b2r):
    return pl.pallas_call(
        _layer1_body,
        grid=(NB,),
        in_specs=[
            pl.BlockSpec((BLK, F1), lambda i: (i, 0)),
            pl.BlockSpec((BLK, 2 * FH), lambda i: (i, 0)),
            pl.BlockSpec((BLK, FT), lambda i: (i, 0)),
            pl.BlockSpec((BLK, FT), lambda i: (i, 0)),
            pl.BlockSpec((8, D), lambda i: (0, 0)),
            pl.BlockSpec((128, D), lambda i: (0, 0)),
            pl.BlockSpec((D, D), lambda i: (0, 0)),
            pl.BlockSpec((8, D), lambda i: (0, 0)),
        ],
        out_specs=[
            pl.BlockSpec((BLK, D), lambda i: (i, 0)),
            pl.BlockSpec((8, D), lambda i: (0, 0)),
        ],
        out_shape=[
            jax.ShapeDtypeStruct((N, D), jnp.float32),
            jax.ShapeDtypeStruct((8, D), jnp.float32),
        ],
    )(x_pad, aggf, e0, e1, b1r, w1_pad, w2, b2r)


def _bn_body(h_ref, st_ref, o_ref):
    mu = st_ref[0:1, :] * (1.0 / N)
    ex2 = st_ref[1:2, :] * (1.0 / N)
    inv = lax.rsqrt(ex2 - mu * mu + 1e-5)
    o_ref[...] = (h_ref[...] - mu) * inv


def _bn_apply(h, stats):
    return pl.pallas_call(
        _bn_body,
        grid=(NB,),
        in_specs=[
            pl.BlockSpec((BLK, D), lambda i: (i, 0)),
            pl.BlockSpec((8, D), lambda i: (0, 0)),
        ],
        out_specs=pl.BlockSpec((BLK, D), lambda i: (i, 0)),
        out_shape=jax.ShapeDtypeStruct((N, D), jnp.float32),
    )(h, stats)


def _layer_body(pool, hn_ref, a0_ref, a1_ref, b1_ref, w1_ref, w2_ref,
                b2_ref, batch_ref, h_ref, st_ref, pooled_ref):
    i = pl.program_id(0)
    z = hn_ref[...] + a0_ref[...] + a1_ref[...]
    zr = jnp.maximum(_bdot(z, w1_ref[...]) + b1_ref[0:1, :], 0.0)
    h = _bdot(zr, w2_ref[...]) + b2_ref[0:1, :]
    h = jnp.maximum(h, 0.0)
    h_ref[...] = h
    _stats_update(st_ref, h, i)

    if pool:
        bb = batch_ref[0, 0, :].reshape(1, BLK)
        gio = lax.broadcasted_iota(jnp.int32, (G, BLK), 0)
        oht = (gio == bb).astype(jnp.float32)
        hx = jnp.concatenate(
            [h, jnp.ones((BLK, 8), jnp.float32)], axis=1)
        pu = jnp.dot(oht, hx, preferred_element_type=jnp.float32,
                     precision=lax.Precision.HIGHEST)

        @pl.when(i == 0)
        def _():
            pooled_ref[...] = jnp.zeros_like(pooled_ref)

        pooled_ref[...] += pu


def _layer(pool, hn, a0, a1, b1r, w1, w2, b2r, batch3d):
    body = functools.partial(_layer_body, pool)
    return pl.pallas_call(
        body,
        grid=(NB,),
        in_specs=[
            pl.BlockSpec((BLK, D), lambda i: (i, 0)),
            pl.BlockSpec((BLK, D), lambda i: (i, 0)),
            pl.BlockSpec((BLK, D), lambda i: (i, 0)),
            pl.BlockSpec((8, D), lambda i: (0, 0)),
            pl.BlockSpec((D, D), lambda i: (0, 0)),
            pl.BlockSpec((D, D), lambda i: (0, 0)),
            pl.BlockSpec((8, D), lambda i: (0, 0)),
            pl.BlockSpec((1, 1, BLK), lambda i: (i, 0, 0)),
        ],
        out_specs=[
            pl.BlockSpec((BLK, D), lambda i: (i, 0)),
            pl.BlockSpec((8, D), lambda i: (0, 0)),
            pl.BlockSpec((G, D + 8), lambda i: (0, 0)),
        ],
        out_shape=[
            jax.ShapeDtypeStruct((N, D), jnp.float32),
            jax.ShapeDtypeStruct((8, D), jnp.float32),
            jax.ShapeDtypeStruct((G, D + 8), jnp.float32),
        ],
    )(hn, a0, a1, b1r, w1, w2, b2r, batch3d)


def _head_body(pooled_ref, st_ref, target_ref, g_ref, bb_ref, n2w_ref,
               n2b_ref, n31w_ref, n31b_ref, n32w_ref, n32b_ref, n4t_ref,
               n4b_ref, n4bias_ref, n5w_ref, n5b_ref, out_ref, xg_ref):
    mu = st_ref[0:1, :] * (1.0 / N)
    ex2 = st_ref[1:2, :] * (1.0 / N)
    inv = lax.rsqrt(ex2 - mu * mu + 1e-5)
    praw = pooled_ref[:, 0:D]
    cnt = pooled_ref[:, D:D + 1]
    pooled = (praw - cnt * mu) * inv

    xg = _bdot(pooled, n2w_ref[...])
    xg = jnp.maximum(xg + n2b_ref[0:1, :], 0.0)
    xg_ref[...] = xg

    t = target_ref[...]
    tm = jnp.mean(t, axis=0, keepdims=True)
    tv = jnp.mean((t - tm) ** 2, axis=0, keepdims=True)
    tn = (t - tm) / jnp.sqrt(tv + 1e-5)
    tn = tn * g_ref[0:1, :] + bb_ref[0:1, :]

    c = _bdot(tn, n31w_ref[...]) + n31b_ref[0:1, :]
    c = _bdot(c, n32w_ref[...]) + n32b_ref[0:1, :]
    c = c - jnp.max(c, axis=1, keepdims=True)
    ec = jnp.exp(c)
    sm = ec / jnp.sum(ec, axis=1, keepdims=True)

    xc = (_bdot(xg, n4t_ref[...]) + _bdot(sm, n4b_ref[...])
          + n4bias_ref[0:1, :])
    xc = jnp.maximum(xc, 0.0)

    o = _bdot(xc, n5w_ref[...])
    o = o[:, 0:1] + n5b_ref[0:1, 0:1]
    out_ref[...] = jax.nn.sigmoid(o)


def _head(pooled, st3, target, g8, b8, n2w, n2b8, n31w, n31b8, n32w, n32b8,
          n4t, n4b, n4bias8, n5wp, n5b8):
    return pl.pallas_call(
        _head_body,
        out_shape=[
            jax.ShapeDtypeStruct((G, 1), jnp.float32),
            jax.ShapeDtypeStruct((G, 128), jnp.float32),
        ],
    )(pooled, st3, target, g8, b8, n2w, n2b8, n31w, n31b8, n32w, n32b8,
      n4t, n4b, n4bias8, n5wp, n5b8)


def _row8(v):
    return jnp.broadcast_to(v.reshape(1, -1), (8, v.shape[0]))


def _pad_edges(v, epad, fill):
    return jnp.concatenate(
        [v, jnp.full((epad - E,), fill, jnp.int32)]).reshape(-1, CH)


def kernel(x, edge_index, batch, target, n11_W1, n11_b1, n11_W2, n11_b2,
           n12_W1, n12_b1, n12_W2, n12_b2, n13_W1, n13_b1, n13_W2, n13_b2,
           n2_W, n2_b, n31_W, n31_b, n32_W, n32_b, n4_W, n4_b, n5_W, n5_b,
           bn1_g, bn1_b):
    src = edge_index[0]
    dst = edge_index[1]
    src2d_w = _pad_edges(src, EPAD_W, 0)
    dst2d_w = _pad_edges(dst, EPAD_W, N)
    src2d_f = _pad_edges(src, EPAD_F, 0)
    dst2d_f = _pad_edges(dst, EPAD_F, N)
    zeros_d = jnp.zeros((RPT, D), jnp.float32)
    zeros_f = jnp.zeros((RPT, FH), jnp.float32)
    zeros_t = jnp.zeros((RPT, FT), jnp.float32)
    x_pad = jnp.pad(x, ((0, 0), (0, F1 - x.shape[1])))
    w1_pad = jnp.pad(n11_W1, ((0, 128 - n11_W1.shape[0]), (0, 0)))
    batch3d = batch.reshape(NB, 1, BLK)

    aggf = _sc_aggregate_x(x_pad[:, :FH], x_pad[:, FH:2 * FH],
                           src2d_f, dst2d_f, zeros_f)
    aggf = jnp.concatenate([aggf[0, :N], aggf[1, :N]], axis=1)
    aggt = _sc_aggregate(x_pad[:, 2 * FH:], src2d_w, dst2d_w, zeros_t,
                         width=FT)
    h1, st1 = _layer1(x_pad, aggf, aggt[0, :N], aggt[1, :N],
                      _row8(n11_b1), w1_pad, n11_W2, _row8(n11_b2))

    h1n = _bn_apply(h1, st1)
    agg2 = _sc_aggregate(h1n, src2d_w, dst2d_w, zeros_d)
    h2, st2, _ = _layer(False, h1n, agg2[0, :N], agg2[1, :N],
                        _row8(n12_b1), n12_W1, n12_W2, _row8(n12_b2),
                        batch3d)

    h2n = _bn_apply(h2, st2)
    agg3 = _sc_aggregate(h2n, src2d_w, dst2d_w, zeros_d)
    _, st3, pooled = _layer(True, h2n, agg3[0, :N], agg3[1, :N],
                            _row8(n13_b1), n13_W1, n13_W2, _row8(n13_b2),
                            batch3d)

    n4t, n4b = n4_W[:128], n4_W[128:]
    n5wp = jnp.pad(n5_W, ((0, 0), (0, 7)))
    out, xg = _head(pooled, st3, target, _row8(bn1_g), _row8(bn1_b),
                    n2_W, _row8(n2_b), n31_W, _row8(n31_b), n32_W,
                    _row8(n32_b), n4t, n4b, _row8(n4_b), n5wp, _row8(n5_b))
    return (out, xg)


# speedup vs baseline: 5.1040x; 1.0006x over previous
import functools

import jax
import jax.numpy as jnp
from jax import lax
from jax.experimental import pallas as pl
from jax.experimental.pallas import tpu as pltpu
from jax.experimental.pallas import tpu_sc as plsc

N = 50000
E = 800000
G = 512
BLK = 2000
NB = N // BLK
D = 32
F1 = 80
FH = 32
FT = 16

NSC = 2
NTILE = 16
NW = NSC * NTILE
CH = 128
K = 8
NPAD = 50048
RPT = NPAD // NTILE

TPT_W = 200
EPAD_W = NW * TPT_W * CH
TPT_F = 392
EPAD_F = NTILE * TPT_F * CH

BF = jnp.bfloat16


def _bdot(a, b):
    return jnp.dot(a.astype(BF), b.astype(BF),
                   preferred_element_type=jnp.float32)


def _edge_loop(table_hbm, src_hbm, dst_hbm, acc_sh, src_v, dst_v, rows_v,
               sems, base, tpt):
    def outer(b, carry):
        row0 = base + b * K
        pltpu.sync_copy(src_hbm.at[pl.ds(row0, K)], src_v)
        pltpu.sync_copy(dst_hbm.at[pl.ds(row0, K)], dst_v)
        descs = [None, None]
        descs[0] = pltpu.async_copy(table_hbm.at[src_v.at[0]], rows_v.at[0],
                                    sems[0])
        for j in range(K):
            if j + 1 < K:
                nb = (j + 1) % 2
                descs[nb] = pltpu.async_copy(
                    table_hbm.at[src_v.at[j + 1]], rows_v.at[nb], sems[nb])
            descs[j % 2].wait()
            pltpu.sync_copy(rows_v.at[j % 2], acc_sh.at[dst_v.at[j]],
                            add=True)
        return carry

    lax.fori_loop(0, tpt // K, outer, 0)


def _sc_agg_edges_body(p_hbm, src_hbm, dst_hbm, zeros_hbm, out_hbm,
                       src_v, dst_v, rows_v, acc_sh, sem0, sem1):
    c = lax.axis_index("c")
    s = lax.axis_index("s")
    wid = c * NTILE + s
    pltpu.sync_copy(zeros_hbm, acc_sh.at[pl.ds(s * RPT, RPT)])
    plsc.subcore_barrier()
    _edge_loop(p_hbm, src_hbm, dst_hbm, acc_sh, src_v, dst_v, rows_v,
               (sem0, sem1), wid * TPT_W, TPT_W)
    plsc.subcore_barrier()
    pltpu.sync_copy(acc_sh.at[pl.ds(s * RPT, RPT)],
                    out_hbm.at[c, pl.ds(s * RPT, RPT)])


def _sc_aggregate(p, src2d, dst2d, zeros_init, width=D):
    mesh = plsc.VectorSubcoreMesh(core_axis_name="c", subcore_axis_name="s")
    f = pl.kernel(
        _sc_agg_edges_body,
        out_type=jax.ShapeDtypeStruct((NSC, NPAD, width), jnp.float32),
        mesh=mesh,
        scratch_types=[
            pltpu.VMEM((K, CH), jnp.int32),
            pltpu.VMEM((K, CH), jnp.int32),
            pltpu.VMEM((2, CH, width), jnp.float32),
            pltpu.VMEM_SHARED((NPAD, width), jnp.float32),
            pltpu.SemaphoreType.DMA,
            pltpu.SemaphoreType.DMA,
        ],
        compiler_params=pltpu.CompilerParams(use_tc_tiling_on_sc=False),
        name="gin_edge_agg",
    )
    return f(p, src2d, dst2d, zeros_init)


def _sc_agg_feat_body(xl_hbm, xr_hbm, src_hbm, dst_hbm, zeros_hbm, out_hbm,
                      src_v, dst_v, rows_v, acc_sh, sem0, sem1):
    c = lax.axis_index("c")
    s = lax.axis_index("s")
    pltpu.sync_copy(zeros_hbm, acc_sh.at[pl.ds(s * RPT, RPT)])
    plsc.subcore_barrier()

    @pl.when(c == 0)
    def _():
        _edge_loop(xl_hbm, src_hbm, dst_hbm, acc_sh, src_v, dst_v, rows_v,
                   (sem0, sem1), s * TPT_F, TPT_F)

    @pl.when(c == 1)
    def _():
        _edge_loop(xr_hbm, src_hbm, dst_hbm, acc_sh, src_v, dst_v, rows_v,
                   (sem0, sem1), s * TPT_F, TPT_F)

    plsc.subcore_barrier()
    pltpu.sync_copy(acc_sh.at[pl.ds(s * RPT, RPT)],
                    out_hbm.at[c, pl.ds(s * RPT, RPT)])


def _sc_aggregate_x(xl, xr, src2d, dst2d, zeros_init):
    mesh = plsc.VectorSubcoreMesh(core_axis_name="c", subcore_axis_name="s")
    f = pl.kernel(
        _sc_agg_feat_body,
        out_type=jax.ShapeDtypeStruct((NSC, NPAD, FH), jnp.float32),
        mesh=mesh,
        scratch_types=[
            pltpu.VMEM((K, CH), jnp.int32),
            pltpu.VMEM((K, CH), jnp.int32),
            pltpu.VMEM((2, CH, FH), jnp.float32),
            pltpu.VMEM_SHARED((NPAD, FH), jnp.float32),
            pltpu.SemaphoreType.DMA,
            pltpu.SemaphoreType.DMA,
        ],
        compiler_params=pltpu.CompilerParams(use_tc_tiling_on_sc=False),
        name="gin_x_agg",
    )
    return f(xl, xr, src2d, dst2d, zeros_init)


def _stats_update(st_ref, h, i):
    s1 = jnp.sum(h, axis=0, keepdims=True)
    s2 = jnp.sum(h * h, axis=0, keepdims=True)
    rows = lax.broadcasted_iota(jnp.int32, (8, D), 0)
    upd = jnp.where(rows == 0, s1, 0.0) + jnp.where(rows == 1, s2, 0.0)

    @pl.when(i == 0)
    def _():
        st_ref[...] = jnp.zeros_like(st_ref)

    st_ref[...] += upd


def _layer1_body(x_ref, af_ref, e0_ref, e1_ref, b1_ref, w1_ref, w2_ref,
                 b2_ref, h_ref, st_ref):
    i = pl.program_id(0)
    a = jnp.concatenate([af_ref[...], e0_ref[...] + e1_ref[...]], axis=1)
    z = x_ref[...] + a
    z = jnp.concatenate([z, jnp.zeros((BLK, 128 - F1), jnp.float32)],
                        axis=1)
    zr = jnp.maximum(_bdot(z, w1_ref[...]) + b1_ref[0:1, :], 0.0)
    h = _bdot(zr, w2_ref[...]) + b2_ref[0:1, :]
    h = jnp.where(h > 0, h, jnp.exp(jnp.minimum(h, 0.0)) - 1.0)
    h_ref[...] = h
    _stats_update(st_ref, h, i)


def _layer1(x_pad, aggf, e0, e1, b1r, w1_pad, w2, b2r):
    return pl.pallas_call(
        _layer1_body,
        grid=(NB,),
        in_specs=[
            pl.BlockSpec((BLK, F1), lambda i: (i, 0)),
            pl.BlockSpec((BLK, 2 * FH), lambda i: (i, 0)),
            pl.BlockSpec((BLK, FT), lambda i: (i, 0)),
            pl.BlockSpec((BLK, FT), lambda i: (i, 0)),
            pl.BlockSpec((8, D), lambda i: (0, 0)),
            pl.BlockSpec((128, D), lambda i: (0, 0)),
            pl.BlockSpec((D, D), lambda i: (0, 0)),
            pl.BlockSpec((8, D), lambda i: (0, 0)),
        ],
        out_specs=[
            pl.BlockSpec((BLK, D), lambda i: (i, 0)),
            pl.BlockSpec((8, D), lambda i: (0, 0)),
        ],
        out_shape=[
            jax.ShapeDtypeStruct((N, D), jnp.float32),
            jax.ShapeDtypeStruct((8, D), jnp.float32),
        ],
    )(x_pad, aggf, e0, e1, b1r, w1_pad, w2, b2r)


def _bn_body(h_ref, st_ref, o_ref):
    mu = st_ref[0:1, :] * (1.0 / N)
    ex2 = st_ref[1:2, :] * (1.0 / N)
    inv = lax.rsqrt(ex2 - mu * mu + 1e-5)
    o_ref[...] = (h_ref[...] - mu) * inv


def _bn_apply(h, stats):
    return pl.pallas_call(
        _bn_body,
        grid=(NB,),
        in_specs=[
            pl.BlockSpec((BLK, D), lambda i: (i, 0)),
            pl.BlockSpec((8, D), lambda i: (0, 0)),
        ],
        out_specs=pl.BlockSpec((BLK, D), lambda i: (i, 0)),
        out_shape=jax.ShapeDtypeStruct((N, D), jnp.float32),
    )(h, stats)


def _layer_body(pool, hn_ref, a0_ref, a1_ref, b1_ref, w1_ref, w2_ref,
                b2_ref, batch_ref, h_ref, st_ref, pooled_ref):
    i = pl.program_id(0)
    z = hn_ref[...] + a0_ref[...] + a1_ref[...]
    zr = jnp.maximum(_bdot(z, w1_ref[...]) + b1_ref[0:1, :], 0.0)
    h = _bdot(zr, w2_ref[...]) + b2_ref[0:1, :]
    h = jnp.maximum(h, 0.0)
    h_ref[...] = h
    _stats_update(st_ref, h, i)

    if pool:
        bb = batch_ref[0, 0, :].reshape(1, BLK)
        gio = lax.broadcasted_iota(jnp.int32, (G, BLK), 0)
        oht = (gio == bb).astype(jnp.float32)
        hx = jnp.concatenate(
            [h, jnp.ones((BLK, 8), jnp.float32)], axis=1)
        pu = jnp.dot(oht, hx, preferred_element_type=jnp.float32,
                     precision=lax.Precision.HIGHEST)

        @pl.when(i == 0)
        def _():
            pooled_ref[...] = jnp.zeros_like(pooled_ref)

        pooled_ref[...] += pu


def _layer(pool, hn, a0, a1, b1r, w1, w2, b2r, batch3d):
    body = functools.partial(_layer_body, pool)
    return pl.pallas_call(
        body,
        grid=(NB,),
        in_specs=[
            pl.BlockSpec((BLK, D), lambda i: (i, 0)),
            pl.BlockSpec((BLK, D), lambda i: (i, 0)),
            pl.BlockSpec((BLK, D), lambda i: (i, 0)),
            pl.BlockSpec((8, D), lambda i: (0, 0)),
            pl.BlockSpec((D, D), lambda i: (0, 0)),
            pl.BlockSpec((D, D), lambda i: (0, 0)),
            pl.BlockSpec((8, D), lambda i: (0, 0)),
            pl.BlockSpec((1, 1, BLK), lambda i: (i, 0, 0)),
        ],
        out_specs=[
            pl.BlockSpec((BLK, D), lambda i: (i, 0)),
            pl.BlockSpec((8, D), lambda i: (0, 0)),
            pl.BlockSpec((G, D + 8), lambda i: (0, 0)),
        ],
        out_shape=[
            jax.ShapeDtypeStruct((N, D), jnp.float32),
            jax.ShapeDtypeStruct((8, D), jnp.float32),
            jax.ShapeDtypeStruct((G, D + 8), jnp.float32),
        ],
    )(hn, a0, a1, b1r, w1, w2, b2r, batch3d)


def _head_body(pooled_ref, st_ref, target_ref, g_ref, bb_ref, n2w_ref,
               n2b_ref, n31w_ref, n31b_ref, n32w_ref, n32b_ref, n4t_ref,
               n4b_ref, n4bias_ref, n5w_ref, n5b_ref, out_ref, xg_ref):
    mu = st_ref[0:1, :] * (1.0 / N)
    ex2 = st_ref[1:2, :] * (1.0 / N)
    inv = lax.rsqrt(ex2 - mu * mu + 1e-5)
    praw = pooled_ref[:, 0:D]
    cnt = pooled_ref[:, D:D + 1]
    pooled = (praw - cnt * mu) * inv

    xg = _bdot(pooled, n2w_ref[...])
    xg = jnp.maximum(xg + n2b_ref[0:1, :], 0.0)
    xg_ref[...] = xg

    t = target_ref[...]
    tm = jnp.mean(t, axis=0, keepdims=True)
    tv = jnp.mean((t - tm) ** 2, axis=0, keepdims=True)
    tn = (t - tm) / jnp.sqrt(tv + 1e-5)
    tn = tn * g_ref[0:1, :] + bb_ref[0:1, :]

    c = _bdot(tn, n31w_ref[...]) + n31b_ref[0:1, :]
    c = _bdot(c, n32w_ref[...]) + n32b_ref[0:1, :]
    c = c - jnp.max(c, axis=1, keepdims=True)
    ec = jnp.exp(c)
    sm = ec / jnp.sum(ec, axis=1, keepdims=True)

    xc = (_bdot(xg, n4t_ref[...]) + _bdot(sm, n4b_ref[...])
          + n4bias_ref[0:1, :])
    xc = jnp.maximum(xc, 0.0)

    o = _bdot(xc, n5w_ref[...])
    o = o[:, 0:1] + n5b_ref[0:1, 0:1]
    out_ref[...] = jax.nn.sigmoid(o)


def _head(pooled, st3, target, g8, b8, n2w, n2b8, n31w, n31b8, n32w, n32b8,
          n4t, n4b, n4bias8, n5wp, n5b8):
    return pl.pallas_call(
        _head_body,
        out_shape=[
            jax.ShapeDtypeStruct((G, 1), jnp.float32),
            jax.ShapeDtypeStruct((G, 128), jnp.float32),
        ],
    )(pooled, st3, target, g8, b8, n2w, n2b8, n31w, n31b8, n32w, n32b8,
      n4t, n4b, n4bias8, n5wp, n5b8)


def _row8(v):
    return jnp.broadcast_to(v.reshape(1, -1), (8, v.shape[0]))


def _pad_edges(v, epad, fill):
    if fill == 0:
        pad = jnp.zeros((epad - E,), jnp.int32)
    else:
        pad = N + (jnp.arange(epad - E, dtype=jnp.int32) % (NPAD - N))
    return jnp.concatenate([v, pad]).reshape(-1, CH)


def kernel(x, edge_index, batch, target, n11_W1, n11_b1, n11_W2, n11_b2,
           n12_W1, n12_b1, n12_W2, n12_b2, n13_W1, n13_b1, n13_W2, n13_b2,
           n2_W, n2_b, n31_W, n31_b, n32_W, n32_b, n4_W, n4_b, n5_W, n5_b,
           bn1_g, bn1_b):
    src = edge_index[0]
    dst = edge_index[1]
    src2d_w = _pad_edges(src, EPAD_W, 0)
    dst2d_w = _pad_edges(dst, EPAD_W, N)
    src2d_f = _pad_edges(src, EPAD_F, 0)
    dst2d_f = _pad_edges(dst, EPAD_F, N)
    zeros_d = jnp.zeros((RPT, D), jnp.float32)
    zeros_f = jnp.zeros((RPT, FH), jnp.float32)
    zeros_t = jnp.zeros((RPT, FT), jnp.float32)
    x_pad = jnp.pad(x, ((0, 0), (0, F1 - x.shape[1])))
    w1_pad = jnp.pad(n11_W1, ((0, 128 - n11_W1.shape[0]), (0, 0)))
    batch3d = batch.reshape(NB, 1, BLK)

    aggf = _sc_aggregate_x(x_pad[:, :FH], x_pad[:, FH:2 * FH],
                           src2d_f, dst2d_f, zeros_f)
    aggf = jnp.concatenate([aggf[0, :N], aggf[1, :N]], axis=1)
    aggt = _sc_aggregate(x_pad[:, 2 * FH:], src2d_w, dst2d_w, zeros_t,
                         width=FT)
    h1, st1 = _layer1(x_pad, aggf, aggt[0, :N], aggt[1, :N],
                      _row8(n11_b1), w1_pad, n11_W2, _row8(n11_b2))

    h1n = _bn_apply(h1, st1)
    agg2 = _sc_aggregate(h1n, src2d_w, dst2d_w, zeros_d)
    h2, st2, _ = _layer(False, h1n, agg2[0, :N], agg2[1, :N],
                        _row8(n12_b1), n12_W1, n12_W2, _row8(n12_b2),
                        batch3d)

    h2n = _bn_apply(h2, st2)
    agg3 = _sc_aggregate(h2n, src2d_w, dst2d_w, zeros_d)
    _, st3, pooled = _layer(True, h2n, agg3[0, :N], agg3[1, :N],
                            _row8(n13_b1), n13_W1, n13_W2, _row8(n13_b2),
                            batch3d)

    n4t, n4b = n4_W[:128], n4_W[128:]
    n5wp = jnp.pad(n5_W, ((0, 0), (0, 7)))
    out, xg = _head(pooled, st3, target, _row8(bn1_g), _row8(bn1_b),
                    n2_W, _row8(n2_b), n31_W, _row8(n31_b), n32_W,
                    _row8(n32_b), n4t, n4b, _row8(n4_b), n5wp, _row8(n5_b))
    return (out, xg)


# speedup vs baseline: 5.1251x; 1.0041x over previous
import functools

import jax
import jax.numpy as jnp
from jax import lax
from jax.experimental import pallas as pl
from jax.experimental.pallas import tpu as pltpu
from jax.experimental.pallas import tpu_sc as plsc

N = 50000
E = 800000
G = 512
BLK = 2000
NB = N // BLK
D = 32
F1 = 80
FH = 32
FT = 16

NSC = 2
NTILE = 16
NW = NSC * NTILE
CH = 128
K = 8
NBUF = 4
NPAD = 50048
RPT = NPAD // NTILE

TPT_W = 200
EPAD_W = NW * TPT_W * CH
TPT_F = 392
EPAD_F = NTILE * TPT_F * CH

BF = jnp.bfloat16


def _bdot(a, b):
    return jnp.dot(a.astype(BF), b.astype(BF),
                   preferred_element_type=jnp.float32)


def _edge_loop(table_hbm, src_hbm, dst_hbm, acc_sh, src_v, dst_v, rows_v,
               gsems, ssem, base, tpt):
    def outer(b, carry):
        row0 = base + b * K
        pltpu.sync_copy(src_hbm.at[pl.ds(row0, K)], src_v)
        pltpu.sync_copy(dst_hbm.at[pl.ds(row0, K)], dst_v)
        for half in range(K // NBUF):
            g = [pltpu.async_copy(
                table_hbm.at[src_v.at[half * NBUF + i]], rows_v.at[i],
                gsems[i]) for i in range(NBUF)]
            s = []
            for i in range(NBUF):
                g[i].wait()
                s.append(pltpu.async_copy(
                    rows_v.at[i], acc_sh.at[dst_v.at[half * NBUF + i]],
                    ssem, add=True))
            for d in s:
                d.wait()
        return carry

    lax.fori_loop(0, tpt // K, outer, 0)


def _sc_agg_edges_body(p_hbm, src_hbm, dst_hbm, zeros_hbm, out_hbm,
                       src_v, dst_v, rows_v, acc_sh, gsems, ssem):
    c = lax.axis_index("c")
    s = lax.axis_index("s")
    wid = c * NTILE + s
    pltpu.sync_copy(zeros_hbm, acc_sh.at[pl.ds(s * RPT, RPT)])
    plsc.subcore_barrier()
    _edge_loop(p_hbm, src_hbm, dst_hbm, acc_sh, src_v, dst_v, rows_v,
               gsems, ssem, wid * TPT_W, TPT_W)
    plsc.subcore_barrier()
    pltpu.sync_copy(acc_sh.at[pl.ds(s * RPT, RPT)],
                    out_hbm.at[c, pl.ds(s * RPT, RPT)])


def _sc_aggregate(p, src2d, dst2d, zeros_init, width=D):
    mesh = plsc.VectorSubcoreMesh(core_axis_name="c", subcore_axis_name="s")
    f = pl.kernel(
        _sc_agg_edges_body,
        out_type=jax.ShapeDtypeStruct((NSC, NPAD, width), jnp.float32),
        mesh=mesh,
        scratch_types=[
            pltpu.VMEM((K, CH), jnp.int32),
            pltpu.VMEM((K, CH), jnp.int32),
            pltpu.VMEM((NBUF, CH, width), jnp.float32),
            pltpu.VMEM_SHARED((NPAD, width), jnp.float32),
            [pltpu.SemaphoreType.DMA] * NBUF,
            pltpu.SemaphoreType.DMA,
        ],
        compiler_params=pltpu.CompilerParams(use_tc_tiling_on_sc=False),
        name="gin_edge_agg",
    )
    return f(p, src2d, dst2d, zeros_init)


def _sc_agg_feat_body(xl_hbm, xr_hbm, src_hbm, dst_hbm, zeros_hbm, out_hbm,
                      src_v, dst_v, rows_v, acc_sh, gsems, ssem):
    c = lax.axis_index("c")
    s = lax.axis_index("s")
    pltpu.sync_copy(zeros_hbm, acc_sh.at[pl.ds(s * RPT, RPT)])
    plsc.subcore_barrier()

    @pl.when(c == 0)
    def _():
        _edge_loop(xl_hbm, src_hbm, dst_hbm, acc_sh, src_v, dst_v, rows_v,
                   gsems, ssem, s * TPT_F, TPT_F)

    @pl.when(c == 1)
    def _():
        _edge_loop(xr_hbm, src_hbm, dst_hbm, acc_sh, src_v, dst_v, rows_v,
                   gsems, ssem, s * TPT_F, TPT_F)

    plsc.subcore_barrier()
    pltpu.sync_copy(acc_sh.at[pl.ds(s * RPT, RPT)],
                    out_hbm.at[c, pl.ds(s * RPT, RPT)])


def _sc_aggregate_x(xl, xr, src2d, dst2d, zeros_init):
    mesh = plsc.VectorSubcoreMesh(core_axis_name="c", subcore_axis_name="s")
    f = pl.kernel(
        _sc_agg_feat_body,
        out_type=jax.ShapeDtypeStruct((NSC, NPAD, FH), jnp.float32),
        mesh=mesh,
        scratch_types=[
            pltpu.VMEM((K, CH), jnp.int32),
            pltpu.VMEM((K, CH), jnp.int32),
            pltpu.VMEM((NBUF, CH, FH), jnp.float32),
            pltpu.VMEM_SHARED((NPAD, FH), jnp.float32),
            [pltpu.SemaphoreType.DMA] * NBUF,
            pltpu.SemaphoreType.DMA,
        ],
        compiler_params=pltpu.CompilerParams(use_tc_tiling_on_sc=False),
        name="gin_x_agg",
    )
    return f(xl, xr, src2d, dst2d, zeros_init)


def _stats_update(st_ref, h, i):
    s1 = jnp.sum(h, axis=0, keepdims=True)
    s2 = jnp.sum(h * h, axis=0, keepdims=True)
    rows = lax.broadcasted_iota(jnp.int32, (8, D), 0)
    upd = jnp.where(rows == 0, s1, 0.0) + jnp.where(rows == 1, s2, 0.0)

    @pl.when(i == 0)
    def _():
        st_ref[...] = jnp.zeros_like(st_ref)

    st_ref[...] += upd


def _layer1_body(x_ref, af_ref, e0_ref, e1_ref, b1_ref, w1_ref, w2_ref,
                 b2_ref, h_ref, st_ref):
    i = pl.program_id(0)
    a = jnp.concatenate([af_ref[...], e0_ref[...] + e1_ref[...]], axis=1)
    z = x_ref[...] + a
    z = jnp.concatenate([z, jnp.zeros((BLK, 128 - F1), jnp.float32)],
                        axis=1)
    zr = jnp.maximum(_bdot(z, w1_ref[...]) + b1_ref[0:1, :], 0.0)
    h = _bdot(zr, w2_ref[...]) + b2_ref[0:1, :]
    h = jnp.where(h > 0, h, jnp.exp(jnp.minimum(h, 0.0)) - 1.0)
    h_ref[...] = h
    _stats_update(st_ref, h, i)


def _layer1(x_pad, aggf, e0, e1, b1r, w1_pad, w2, b2r):
    return pl.pallas_call(
        _layer1_body,
        grid=(NB,),
        in_specs=[
            pl.BlockSpec((BLK, F1), lambda i: (i, 0)),
            pl.BlockSpec((BLK, 2 * FH), lambda i: (i, 0)),
            pl.BlockSpec((BLK, FT), lambda i: (i, 0)),
            pl.BlockSpec((BLK, FT), lambda i: (i, 0)),
            pl.BlockSpec((8, D), lambda i: (0, 0)),
            pl.BlockSpec((128, D), lambda i: (0, 0)),
            pl.BlockSpec((D, D), lambda i: (0, 0)),
            pl.BlockSpec((8, D), lambda i: (0, 0)),
        ],
        out_specs=[
            pl.BlockSpec((BLK, D), lambda i: (i, 0)),
            pl.BlockSpec((8, D), lambda i: (0, 0)),
        ],
        out_shape=[
            jax.ShapeDtypeStruct((N, D), jnp.float32),
            jax.ShapeDtypeStruct((8, D), jnp.float32),
        ],
    )(x_pad, aggf, e0, e1, b1r, w1_pad, w2, b2r)


def _bn_body(h_ref, st_ref, o_ref):
    mu = st_ref[0:1, :] * (1.0 / N)
    ex2 = st_ref[1:2, :] * (1.0 / N)
    inv = lax.rsqrt(ex2 - mu * mu + 1e-5)
    o_ref[...] = (h_ref[...] - mu) * inv


def _bn_apply(h, stats):
    return pl.pallas_call(
        _bn_body,
        grid=(NB,),
        in_specs=[
            pl.BlockSpec((BLK, D), lambda i: (i, 0)),
            pl.BlockSpec((8, D), lambda i: (0, 0)),
        ],
        out_specs=pl.BlockSpec((BLK, D), lambda i: (i, 0)),
        out_shape=jax.ShapeDtypeStruct((N, D), jnp.float32),
    )(h, stats)


def _layer_body(pool, hn_ref, a0_ref, a1_ref, b1_ref, w1_ref, w2_ref,
                b2_ref, batch_ref, h_ref, st_ref, pooled_ref):
    i = pl.program_id(0)
    z = hn_ref[...] + a0_ref[...] + a1_ref[...]
    zr = jnp.maximum(_bdot(z, w1_ref[...]) + b1_ref[0:1, :], 0.0)
    h = _bdot(zr, w2_ref[...]) + b2_ref[0:1, :]
    h = jnp.maximum(h, 0.0)
    h_ref[...] = h
    _stats_update(st_ref, h, i)

    if pool:
        bb = batch_ref[0, 0, :].reshape(1, BLK)
        gio = lax.broadcasted_iota(jnp.int32, (G, BLK), 0)
        oht = (gio == bb).astype(jnp.float32)
        hx = jnp.concatenate(
            [h, jnp.ones((BLK, 8), jnp.float32)], axis=1)
        pu = jnp.dot(oht, hx, preferred_element_type=jnp.float32,
                     precision=lax.Precision.HIGHEST)

        @pl.when(i == 0)
        def _():
            pooled_ref[...] = jnp.zeros_like(pooled_ref)

        pooled_ref[...] += pu


def _layer(pool, hn, a0, a1, b1r, w1, w2, b2r, batch3d):
    body = functools.partial(_layer_body, pool)
    return pl.pallas_call(
        body,
        grid=(NB,),
        in_specs=[
            pl.BlockSpec((BLK, D), lambda i: (i, 0)),
            pl.BlockSpec((BLK, D), lambda i: (i, 0)),
            pl.BlockSpec((BLK, D), lambda i: (i, 0)),
            pl.BlockSpec((8, D), lambda i: (0, 0)),
            pl.BlockSpec((D, D), lambda i: (0, 0)),
            pl.BlockSpec((D, D), lambda i: (0, 0)),
            pl.BlockSpec((8, D), lambda i: (0, 0)),
            pl.BlockSpec((1, 1, BLK), lambda i: (i, 0, 0)),
        ],
        out_specs=[
            pl.BlockSpec((BLK, D), lambda i: (i, 0)),
            pl.BlockSpec((8, D), lambda i: (0, 0)),
            pl.BlockSpec((G, D + 8), lambda i: (0, 0)),
        ],
        out_shape=[
            jax.ShapeDtypeStruct((N, D), jnp.float32),
            jax.ShapeDtypeStruct((8, D), jnp.float32),
            jax.ShapeDtypeStruct((G, D + 8), jnp.float32),
        ],
    )(hn, a0, a1, b1r, w1, w2, b2r, batch3d)


def _head_body(pooled_ref, st_ref, target_ref, g_ref, bb_ref, n2w_ref,
               n2b_ref, n31w_ref, n31b_ref, n32w_ref, n32b_ref, n4t_ref,
               n4b_ref, n4bias_ref, n5w_ref, n5b_ref, out_ref, xg_ref):
    mu = st_ref[0:1, :] * (1.0 / N)
    ex2 = st_ref[1:2, :] * (1.0 / N)
    inv = lax.rsqrt(ex2 - mu * mu + 1e-5)
    praw = pooled_ref[:, 0:D]
    cnt = pooled_ref[:, D:D + 1]
    pooled = (praw - cnt * mu) * inv

    xg = _bdot(pooled, n2w_ref[...])
    xg = jnp.maximum(xg + n2b_ref[0:1, :], 0.0)
    xg_ref[...] = xg

    t = target_ref[...]
    tm = jnp.mean(t, axis=0, keepdims=True)
    tv = jnp.mean((t - tm) ** 2, axis=0, keepdims=True)
    tn = (t - tm) / jnp.sqrt(tv + 1e-5)
    tn = tn * g_ref[0:1, :] + bb_ref[0:1, :]

    c = _bdot(tn, n31w_ref[...]) + n31b_ref[0:1, :]
    c = _bdot(c, n32w_ref[...]) + n32b_ref[0:1, :]
    c = c - jnp.max(c, axis=1, keepdims=True)
    ec = jnp.exp(c)
    sm = ec / jnp.sum(ec, axis=1, keepdims=True)

    xc = (_bdot(xg, n4t_ref[...]) + _bdot(sm, n4b_ref[...])
          + n4bias_ref[0:1, :])
    xc = jnp.maximum(xc, 0.0)

    o = _bdot(xc, n5w_ref[...])
    o = o[:, 0:1] + n5b_ref[0:1, 0:1]
    out_ref[...] = jax.nn.sigmoid(o)


def _head(pooled, st3, target, g8, b8, n2w, n2b8, n31w, n31b8, n32w, n32b8,
          n4t, n4b, n4bias8, n5wp, n5b8):
    return pl.pallas_call(
        _head_body,
        out_shape=[
            jax.ShapeDtypeStruct((G, 1), jnp.float32),
            jax.ShapeDtypeStruct((G, 128), jnp.float32),
        ],
    )(pooled, st3, target, g8, b8, n2w, n2b8, n31w, n31b8, n32w, n32b8,
      n4t, n4b, n4bias8, n5wp, n5b8)


def _row8(v):
    return jnp.broadcast_to(v.reshape(1, -1), (8, v.shape[0]))


def _pad_edges(v, epad, fill):
    if fill == 0:
        pad = jnp.zeros((epad - E,), jnp.int32)
    else:
        pad = N + (jnp.arange(epad - E, dtype=jnp.int32) % (NPAD - N))
    return jnp.concatenate([v, pad]).reshape(-1, CH)


def kernel(x, edge_index, batch, target, n11_W1, n11_b1, n11_W2, n11_b2,
           n12_W1, n12_b1, n12_W2, n12_b2, n13_W1, n13_b1, n13_W2, n13_b2,
           n2_W, n2_b, n31_W, n31_b, n32_W, n32_b, n4_W, n4_b, n5_W, n5_b,
           bn1_g, bn1_b):
    src = edge_index[0]
    dst = edge_index[1]
    src2d_w = _pad_edges(src, EPAD_W, 0)
    dst2d_w = _pad_edges(dst, EPAD_W, N)
    src2d_f = _pad_edges(src, EPAD_F, 0)
    dst2d_f = _pad_edges(dst, EPAD_F, N)
    zeros_d = jnp.zeros((RPT, D), jnp.float32)
    zeros_f = jnp.zeros((RPT, FH), jnp.float32)
    zeros_t = jnp.zeros((RPT, FT), jnp.float32)
    x_pad = jnp.pad(x, ((0, 0), (0, F1 - x.shape[1])))
    w1_pad = jnp.pad(n11_W1, ((0, 128 - n11_W1.shape[0]), (0, 0)))
    batch3d = batch.reshape(NB, 1, BLK)

    aggf = _sc_aggregate_x(x_pad[:, :FH], x_pad[:, FH:2 * FH],
                           src2d_f, dst2d_f, zeros_f)
    aggf = jnp.concatenate([aggf[0, :N], aggf[1, :N]], axis=1)
    aggt = _sc_aggregate(x_pad[:, 2 * FH:], src2d_w, dst2d_w, zeros_t,
                         width=FT)
    h1, st1 = _layer1(x_pad, aggf, aggt[0, :N], aggt[1, :N],
                      _row8(n11_b1), w1_pad, n11_W2, _row8(n11_b2))

    h1n = _bn_apply(h1, st1)
    agg2 = _sc_aggregate(h1n, src2d_w, dst2d_w, zeros_d)
    h2, st2, _ = _layer(False, h1n, agg2[0, :N], agg2[1, :N],
                        _row8(n12_b1), n12_W1, n12_W2, _row8(n12_b2),
                        batch3d)

    h2n = _bn_apply(h2, st2)
    agg3 = _sc_aggregate(h2n, src2d_w, dst2d_w, zeros_d)
    _, st3, pooled = _layer(True, h2n, agg3[0, :N], agg3[1, :N],
                            _row8(n13_b1), n13_W1, n13_W2, _row8(n13_b2),
                            batch3d)

    n4t, n4b = n4_W[:128], n4_W[128:]
    n5wp = jnp.pad(n5_W, ((0, 0), (0, 7)))
    out, xg = _head(pooled, st3, target, _row8(bn1_g), _row8(bn1_b),
                    n2_W, _row8(n2_b), n31_W, _row8(n31_b), n32_W,
                    _row8(n32_b), n4t, n4b, _row8(n4_b), n5wp, _row8(n5_b))
    return (out, xg)


# speedup vs baseline: 5.7036x; 1.1129x over previous
import functools

import jax
import jax.numpy as jnp
from jax import lax
from jax.experimental import pallas as pl
from jax.experimental.pallas import tpu as pltpu
from jax.experimental.pallas import tpu_sc as plsc

N = 50000
E = 800000
G = 512
BLK = 2000
NB = N // BLK
D = 32
F1 = 80
FH = 32
FT = 16

NSC = 2
NTILE = 16
NW = NSC * NTILE
CH = 128
K = 8
NBUF = 4
NPAD = 50048
RPT = NPAD // NTILE

TPT_W = 200
EPAD_W = NW * TPT_W * CH
TPT_F = 392
EPAD_F = NTILE * TPT_F * CH

BF = jnp.bfloat16


def _bdot(a, b):
    return jnp.dot(a.astype(BF), b.astype(BF),
                   preferred_element_type=jnp.float32)


def _edge_loop(table_hbm, src_hbm, dst_hbm, acc_sh, src_v, dst_v, rows_v,
               gsems, ssem, base, tpt):
    def outer(b, carry):
        row0 = base + b * K
        pltpu.sync_copy(src_hbm.at[pl.ds(row0, K)], src_v)
        pltpu.sync_copy(dst_hbm.at[pl.ds(row0, K)], dst_v)
        for half in range(K // NBUF):
            g = [pltpu.async_copy(
                table_hbm.at[src_v.at[half * NBUF + i]], rows_v.at[i],
                gsems[i]) for i in range(NBUF)]
            s = []
            for i in range(NBUF):
                g[i].wait()
                s.append(pltpu.async_copy(
                    rows_v.at[i], acc_sh.at[dst_v.at[half * NBUF + i]],
                    ssem, add=True))
            for d in s:
                d.wait()
        return carry

    lax.fori_loop(0, tpt // K, outer, 0)


def _sc_agg_edges_body(p_hbm, src_hbm, dst_hbm, zeros_hbm, out_hbm,
                       src_v, dst_v, rows_v, acc_sh, gsems, ssem):
    c = lax.axis_index("c")
    s = lax.axis_index("s")
    wid = c * NTILE + s
    pltpu.sync_copy(zeros_hbm, acc_sh.at[pl.ds(s * RPT, RPT)])
    plsc.subcore_barrier()
    _edge_loop(p_hbm, src_hbm, dst_hbm, acc_sh, src_v, dst_v, rows_v,
               gsems, ssem, wid * TPT_W, TPT_W)
    plsc.subcore_barrier()
    pltpu.sync_copy(acc_sh.at[pl.ds(s * RPT, RPT)],
                    out_hbm.at[c, pl.ds(s * RPT, RPT)])


def _sc_aggregate(p, src2d, dst2d, zeros_init, width=D):
    mesh = plsc.VectorSubcoreMesh(core_axis_name="c", subcore_axis_name="s")
    f = pl.kernel(
        _sc_agg_edges_body,
        out_type=jax.ShapeDtypeStruct((NSC, NPAD, width), jnp.float32),
        mesh=mesh,
        scratch_types=[
            pltpu.VMEM((K, CH), jnp.int32),
            pltpu.VMEM((K, CH), jnp.int32),
            pltpu.VMEM((NBUF, CH, width), jnp.float32),
            pltpu.VMEM_SHARED((NPAD, width), jnp.float32),
            [pltpu.SemaphoreType.DMA] * NBUF,
            pltpu.SemaphoreType.DMA,
        ],
        compiler_params=pltpu.CompilerParams(use_tc_tiling_on_sc=False),
        name="gin_edge_agg",
    )
    return f(p, src2d, dst2d, zeros_init)


def _sc_agg_feat_body(xl_hbm, xr_hbm, src_hbm, dst_hbm, zeros_hbm, out_hbm,
                      src_v, dst_v, rows_v, acc_sh, gsems, ssem):
    c = lax.axis_index("c")
    s = lax.axis_index("s")
    pltpu.sync_copy(zeros_hbm, acc_sh.at[pl.ds(s * RPT, RPT)])
    plsc.subcore_barrier()

    @pl.when(c == 0)
    def _():
        _edge_loop(xl_hbm, src_hbm, dst_hbm, acc_sh, src_v, dst_v, rows_v,
                   gsems, ssem, s * TPT_F, TPT_F)

    @pl.when(c == 1)
    def _():
        _edge_loop(xr_hbm, src_hbm, dst_hbm, acc_sh, src_v, dst_v, rows_v,
                   gsems, ssem, s * TPT_F, TPT_F)

    plsc.subcore_barrier()
    pltpu.sync_copy(acc_sh.at[pl.ds(s * RPT, RPT)],
                    out_hbm.at[c, pl.ds(s * RPT, RPT)])


def _sc_aggregate_x(xl, xr, src2d, dst2d, zeros_init):
    mesh = plsc.VectorSubcoreMesh(core_axis_name="c", subcore_axis_name="s")
    f = pl.kernel(
        _sc_agg_feat_body,
        out_type=jax.ShapeDtypeStruct((NSC, NPAD, FH), jnp.float32),
        mesh=mesh,
        scratch_types=[
            pltpu.VMEM((K, CH), jnp.int32),
            pltpu.VMEM((K, CH), jnp.int32),
            pltpu.VMEM((NBUF, CH, FH), jnp.float32),
            pltpu.VMEM_SHARED((NPAD, FH), jnp.float32),
            [pltpu.SemaphoreType.DMA] * NBUF,
            pltpu.SemaphoreType.DMA,
        ],
        compiler_params=pltpu.CompilerParams(use_tc_tiling_on_sc=False),
        name="gin_x_agg",
    )
    return f(xl, xr, src2d, dst2d, zeros_init)


def _stats_update(st_ref, h, i):
    s1 = jnp.sum(h, axis=0, keepdims=True)
    s2 = jnp.sum(h * h, axis=0, keepdims=True)
    rows = lax.broadcasted_iota(jnp.int32, (8, D), 0)
    upd = jnp.where(rows == 0, s1, 0.0) + jnp.where(rows == 1, s2, 0.0)

    @pl.when(i == 0)
    def _():
        st_ref[...] = jnp.zeros_like(st_ref)

    st_ref[...] += upd


def _layer1_body(x_ref, af_ref, e0_ref, e1_ref, b1_ref, w1_ref, w2_ref,
                 b2_ref, h_ref, st_ref):
    i = pl.program_id(0)
    a = jnp.concatenate([af_ref[...], e0_ref[...] + e1_ref[...]], axis=1)
    z = x_ref[...] + a
    z = jnp.concatenate([z, jnp.zeros((BLK, 128 - F1), jnp.float32)],
                        axis=1)
    zr = jnp.maximum(_bdot(z, w1_ref[...]) + b1_ref[0:1, :], 0.0)
    h = _bdot(zr, w2_ref[...]) + b2_ref[0:1, :]
    h = jnp.where(h > 0, h, jnp.exp(jnp.minimum(h, 0.0)) - 1.0)
    h_ref[...] = h
    _stats_update(st_ref, h, i)


def _layer1(x_pad, aggf, e0, e1, b1r, w1_pad, w2, b2r):
    return pl.pallas_call(
        _layer1_body,
        grid=(NB,),
        in_specs=[
            pl.BlockSpec((BLK, F1), lambda i: (i, 0)),
            pl.BlockSpec((BLK, 2 * FH), lambda i: (i, 0)),
            pl.BlockSpec((BLK, FT), lambda i: (i, 0)),
            pl.BlockSpec((BLK, FT), lambda i: (i, 0)),
            pl.BlockSpec((8, D), lambda i: (0, 0)),
            pl.BlockSpec((128, D), lambda i: (0, 0)),
            pl.BlockSpec((D, D), lambda i: (0, 0)),
            pl.BlockSpec((8, D), lambda i: (0, 0)),
        ],
        out_specs=[
            pl.BlockSpec((BLK, D), lambda i: (i, 0)),
            pl.BlockSpec((8, D), lambda i: (0, 0)),
        ],
        out_shape=[
            jax.ShapeDtypeStruct((N, D), jnp.float32),
            jax.ShapeDtypeStruct((8, D), jnp.float32),
        ],
    )(x_pad, aggf, e0, e1, b1r, w1_pad, w2, b2r)


def _bn_body(h_ref, st_ref, o_ref):
    mu = st_ref[0:1, :] * (1.0 / N)
    ex2 = st_ref[1:2, :] * (1.0 / N)
    inv = lax.rsqrt(ex2 - mu * mu + 1e-5)
    o_ref[...] = (h_ref[...] - mu) * inv


def _bn_apply(h, stats):
    return pl.pallas_call(
        _bn_body,
        grid=(NB,),
        in_specs=[
            pl.BlockSpec((BLK, D), lambda i: (i, 0)),
            pl.BlockSpec((8, D), lambda i: (0, 0)),
        ],
        out_specs=pl.BlockSpec((BLK, D), lambda i: (i, 0)),
        out_shape=jax.ShapeDtypeStruct((N, D), jnp.float32),
    )(h, stats)


def _layer_body(pool, hn_ref, a0_ref, a1_ref, b1_ref, w1_ref, w2_ref,
                b2_ref, batch_ref, h_ref, st_ref, pooled_ref):
    i = pl.program_id(0)
    z = hn_ref[...] + a0_ref[...] + a1_ref[...]
    zr = jnp.maximum(_bdot(z, w1_ref[...]) + b1_ref[0:1, :], 0.0)
    h = _bdot(zr, w2_ref[...]) + b2_ref[0:1, :]
    h = jnp.maximum(h, 0.0)
    h_ref[...] = h
    _stats_update(st_ref, h, i)

    if pool:
        bb = batch_ref[0, 0, :].reshape(1, BLK)
        gio = lax.broadcasted_iota(jnp.int32, (G, BLK), 0)
        oht = (gio == bb).astype(jnp.float32)
        hx = jnp.concatenate(
            [h, jnp.ones((BLK, 8), jnp.float32)], axis=1)
        pu = jnp.dot(oht, hx, preferred_element_type=jnp.float32,
                     precision=lax.Precision.HIGHEST)

        @pl.when(i == 0)
        def _():
            pooled_ref[...] = jnp.zeros_like(pooled_ref)

        pooled_ref[...] += pu


def _layer(pool, hn, a0, a1, b1r, w1, w2, b2r, batch3d):
    body = functools.partial(_layer_body, pool)
    return pl.pallas_call(
        body,
        grid=(NB,),
        in_specs=[
            pl.BlockSpec((BLK, D), lambda i: (i, 0)),
            pl.BlockSpec((BLK, D), lambda i: (i, 0)),
            pl.BlockSpec((BLK, D), lambda i: (i, 0)),
            pl.BlockSpec((8, D), lambda i: (0, 0)),
            pl.BlockSpec((D, D), lambda i: (0, 0)),
            pl.BlockSpec((D, D), lambda i: (0, 0)),
            pl.BlockSpec((8, D), lambda i: (0, 0)),
            pl.BlockSpec((1, 1, BLK), lambda i: (i, 0, 0)),
        ],
        out_specs=[
            pl.BlockSpec((BLK, D), lambda i: (i, 0)),
            pl.BlockSpec((8, D), lambda i: (0, 0)),
            pl.BlockSpec((G, D + 8), lambda i: (0, 0)),
        ],
        out_shape=[
            jax.ShapeDtypeStruct((N, D), jnp.float32),
            jax.ShapeDtypeStruct((8, D), jnp.float32),
            jax.ShapeDtypeStruct((G, D + 8), jnp.float32),
        ],
    )(hn, a0, a1, b1r, w1, w2, b2r, batch3d)


def _head_body(pooled_ref, st_ref, target_ref, g_ref, bb_ref, n2w_ref,
               n2b_ref, n31w_ref, n31b_ref, n32w_ref, n32b_ref, n4t_ref,
               n4b_ref, n4bias_ref, n5w_ref, n5b_ref, out_ref, xg_ref):
    mu = st_ref[0:1, :] * (1.0 / N)
    ex2 = st_ref[1:2, :] * (1.0 / N)
    inv = lax.rsqrt(ex2 - mu * mu + 1e-5)
    praw = pooled_ref[:, 0:D]
    cnt = pooled_ref[:, D:D + 1]
    pooled = (praw - cnt * mu) * inv

    xg = _bdot(pooled, n2w_ref[...])
    xg = jnp.maximum(xg + n2b_ref[0:1, :], 0.0)
    xg_ref[...] = xg

    t = target_ref[...]
    tm = jnp.mean(t, axis=0, keepdims=True)
    tv = jnp.mean((t - tm) ** 2, axis=0, keepdims=True)
    tn = (t - tm) / jnp.sqrt(tv + 1e-5)
    tn = tn * g_ref[0:1, :] + bb_ref[0:1, :]

    c = _bdot(tn, n31w_ref[...]) + n31b_ref[0:1, :]
    c = _bdot(c, n32w_ref[...]) + n32b_ref[0:1, :]
    c = c - jnp.max(c, axis=1, keepdims=True)
    ec = jnp.exp(c)
    sm = ec / jnp.sum(ec, axis=1, keepdims=True)

    xc = (_bdot(xg, n4t_ref[...]) + _bdot(sm, n4b_ref[...])
          + n4bias_ref[0:1, :])
    xc = jnp.maximum(xc, 0.0)

    o = _bdot(xc, n5w_ref[...])
    o = o[:, 0:1] + n5b_ref[0:1, 0:1]
    out_ref[...] = jax.nn.sigmoid(o)


def _head(pooled, st3, target, g8, b8, n2w, n2b8, n31w, n31b8, n32w, n32b8,
          n4t, n4b, n4bias8, n5wp, n5b8):
    return pl.pallas_call(
        _head_body,
        out_shape=[
            jax.ShapeDtypeStruct((G, 1), jnp.float32),
            jax.ShapeDtypeStruct((G, 128), jnp.float32),
        ],
    )(pooled, st3, target, g8, b8, n2w, n2b8, n31w, n31b8, n32w, n32b8,
      n4t, n4b, n4bias8, n5wp, n5b8)


def _row8(v):
    return jnp.broadcast_to(v.reshape(1, -1), (8, v.shape[0]))


def _pad_edges(v, ntiles, tpt, fill):
    q = E // ntiles
    p = tpt * CH - q
    vt = v.reshape(ntiles, q)
    if fill == 0:
        pad = jnp.zeros((ntiles, p), jnp.int32)
    else:
        pad = jnp.broadcast_to(
            N + (jnp.arange(p, dtype=jnp.int32) % (NPAD - N)), (ntiles, p))
    return jnp.concatenate([vt, pad], axis=1).reshape(-1, CH)


def kernel(x, edge_index, batch, target, n11_W1, n11_b1, n11_W2, n11_b2,
           n12_W1, n12_b1, n12_W2, n12_b2, n13_W1, n13_b1, n13_W2, n13_b2,
           n2_W, n2_b, n31_W, n31_b, n32_W, n32_b, n4_W, n4_b, n5_W, n5_b,
           bn1_g, bn1_b):
    src = edge_index[0]
    dst = edge_index[1]
    src2d_w = _pad_edges(src, NW, TPT_W, 0)
    dst2d_w = _pad_edges(dst, NW, TPT_W, N)
    src2d_f = _pad_edges(src, NTILE, TPT_F, 0)
    dst2d_f = _pad_edges(dst, NTILE, TPT_F, N)
    zeros_d = jnp.zeros((RPT, D), jnp.float32)
    zeros_f = jnp.zeros((RPT, FH), jnp.float32)
    zeros_t = jnp.zeros((RPT, FT), jnp.float32)
    x_pad = jnp.pad(x, ((0, 0), (0, F1 - x.shape[1])))
    w1_pad = jnp.pad(n11_W1, ((0, 128 - n11_W1.shape[0]), (0, 0)))
    batch3d = batch.reshape(NB, 1, BLK)

    aggf = _sc_aggregate_x(x_pad[:, :FH], x_pad[:, FH:2 * FH],
                           src2d_f, dst2d_f, zeros_f)
    aggf = jnp.concatenate([aggf[0, :N], aggf[1, :N]], axis=1)
    aggt = _sc_aggregate(x_pad[:, 2 * FH:], src2d_w, dst2d_w, zeros_t,
                         width=FT)
    h1, st1 = _layer1(x_pad, aggf, aggt[0, :N], aggt[1, :N],
                      _row8(n11_b1), w1_pad, n11_W2, _row8(n11_b2))

    h1n = _bn_apply(h1, st1)
    agg2 = _sc_aggregate(h1n, src2d_w, dst2d_w, zeros_d)
    h2, st2, _ = _layer(False, h1n, agg2[0, :N], agg2[1, :N],
                        _row8(n12_b1), n12_W1, n12_W2, _row8(n12_b2),
                        batch3d)

    h2n = _bn_apply(h2, st2)
    agg3 = _sc_aggregate(h2n, src2d_w, dst2d_w, zeros_d)
    _, st3, pooled = _layer(True, h2n, agg3[0, :N], agg3[1, :N],
                            _row8(n13_b1), n13_W1, n13_W2, _row8(n13_b2),
                            batch3d)

    n4t, n4b = n4_W[:128], n4_W[128:]
    n5wp = jnp.pad(n5_W, ((0, 0), (0, 7)))
    out, xg = _head(pooled, st3, target, _row8(bn1_g), _row8(bn1_b),
                    n2_W, _row8(n2_b), n31_W, _row8(n31_b), n32_W,
                    _row8(n32_b), n4t, n4b, _row8(n4_b), n5wp, _row8(n5_b))
    return (out, xg)


# speedup vs baseline: 6.9624x; 1.2207x over previous
import functools

import jax
import jax.numpy as jnp
from jax import lax
from jax.experimental import pallas as pl
from jax.experimental.pallas import tpu as pltpu
from jax.experimental.pallas import tpu_sc as plsc

N = 50000
E = 800000
G = 512
BLK = 2000
NB = N // BLK
D = 32
F1 = 80
FH = 32
FT = 16

NSC = 2
NTILE = 16
NW = NSC * NTILE
CH = 128
K = 3
BSZ = K * CH
NPAD = 50048
RPT = NPAD // NTILE

BPT_W = 66
EPAD_W = NW * BPT_W * BSZ
BPT_F = 131
EPAD_F = NTILE * BPT_F * BSZ

BF = jnp.bfloat16


def _bdot(a, b):
    return jnp.dot(a.astype(BF), b.astype(BF),
                   preferred_element_type=jnp.float32)


def _edge_loop(table_hbm, src_hbm, dst_hbm, acc_sh, src_v, dst_v, rows_v,
               gsems, ssems, isems, base, tpt):
    blocks = tpt
    g = {}
    s = {}
    i = {}
    pltpu.sync_copy(src_hbm.at[base], src_v.at[0])
    pltpu.sync_copy(dst_hbm.at[base], dst_v.at[0])
    for b in range(blocks):
        bb = b % 2
        if b >= 2:
            s[b - 2].wait()
        if b >= 1:
            for d in i[b]:
                d.wait()
        g[b] = pltpu.async_copy(table_hbm.at[src_v.at[bb]], rows_v.at[bb],
                                gsems[bb])
        if b + 1 < blocks:
            nb = (b + 1) % 2
            i[b + 1] = (
                pltpu.async_copy(src_hbm.at[base + b + 1], src_v.at[nb],
                                 isems[nb]),
                pltpu.async_copy(dst_hbm.at[base + b + 1], dst_v.at[nb],
                                 isems[nb]),
            )
        g[b].wait()
        s[b] = pltpu.async_copy(rows_v.at[bb], acc_sh.at[dst_v.at[bb]],
                                ssems[bb], add=True)
    if blocks >= 2:
        s[blocks - 2].wait()
    s[blocks - 1].wait()


def _sc_agg_edges_body(p_hbm, src_hbm, dst_hbm, zeros_hbm, out_hbm,
                       src_v, dst_v, rows_v, acc_sh, gsems, ssems, isems):
    c = lax.axis_index("c")
    s = lax.axis_index("s")
    wid = c * NTILE + s
    pltpu.sync_copy(zeros_hbm, acc_sh.at[pl.ds(s * RPT, RPT)])
    plsc.subcore_barrier()
    _edge_loop(p_hbm, src_hbm, dst_hbm, acc_sh, src_v, dst_v, rows_v,
               gsems, ssems, isems, wid * BPT_W, BPT_W)
    plsc.subcore_barrier()
    pltpu.sync_copy(acc_sh.at[pl.ds(s * RPT, RPT)],
                    out_hbm.at[c, pl.ds(s * RPT, RPT)])


def _sc_aggregate(p, src2d, dst2d, zeros_init, width=D):
    mesh = plsc.VectorSubcoreMesh(core_axis_name="c", subcore_axis_name="s")
    f = pl.kernel(
        _sc_agg_edges_body,
        out_type=jax.ShapeDtypeStruct((NSC, NPAD, width), jnp.float32),
        mesh=mesh,
        scratch_types=[
            pltpu.VMEM((2, BSZ), jnp.int32),
            pltpu.VMEM((2, BSZ), jnp.int32),
            pltpu.VMEM((2, BSZ, width), jnp.float32),
            pltpu.VMEM_SHARED((NPAD, width), jnp.float32),
            [pltpu.SemaphoreType.DMA] * 2,
            [pltpu.SemaphoreType.DMA] * 2,
            [pltpu.SemaphoreType.DMA] * 2,
        ],
        compiler_params=pltpu.CompilerParams(use_tc_tiling_on_sc=False),
        name="gin_edge_agg",
    )
    return f(p, src2d, dst2d, zeros_init)


def _sc_agg_feat_body(xl_hbm, xr_hbm, src_hbm, dst_hbm, zeros_hbm, out_hbm,
                      src_v, dst_v, rows_v, acc_sh, gsems, ssems, isems):
    c = lax.axis_index("c")
    s = lax.axis_index("s")
    pltpu.sync_copy(zeros_hbm, acc_sh.at[pl.ds(s * RPT, RPT)])
    plsc.subcore_barrier()

    @pl.when(c == 0)
    def _():
        _edge_loop(xl_hbm, src_hbm, dst_hbm, acc_sh, src_v, dst_v, rows_v,
                   gsems, ssems, isems, s * BPT_F, BPT_F)

    @pl.when(c == 1)
    def _():
        _edge_loop(xr_hbm, src_hbm, dst_hbm, acc_sh, src_v, dst_v, rows_v,
                   gsems, ssems, isems, s * BPT_F, BPT_F)

    plsc.subcore_barrier()
    pltpu.sync_copy(acc_sh.at[pl.ds(s * RPT, RPT)],
                    out_hbm.at[c, pl.ds(s * RPT, RPT)])


def _sc_aggregate_x(xl, xr, src2d, dst2d, zeros_init):
    mesh = plsc.VectorSubcoreMesh(core_axis_name="c", subcore_axis_name="s")
    f = pl.kernel(
        _sc_agg_feat_body,
        out_type=jax.ShapeDtypeStruct((NSC, NPAD, FH), jnp.float32),
        mesh=mesh,
        scratch_types=[
            pltpu.VMEM((2, BSZ), jnp.int32),
            pltpu.VMEM((2, BSZ), jnp.int32),
            pltpu.VMEM((2, BSZ, FH), jnp.float32),
            pltpu.VMEM_SHARED((NPAD, FH), jnp.float32),
            [pltpu.SemaphoreType.DMA] * 2,
            [pltpu.SemaphoreType.DMA] * 2,
            [pltpu.SemaphoreType.DMA] * 2,
        ],
        compiler_params=pltpu.CompilerParams(use_tc_tiling_on_sc=False),
        name="gin_x_agg",
    )
    return f(xl, xr, src2d, dst2d, zeros_init)


def _stats_update(st_ref, h, i):
    s1 = jnp.sum(h, axis=0, keepdims=True)
    s2 = jnp.sum(h * h, axis=0, keepdims=True)
    rows = lax.broadcasted_iota(jnp.int32, (8, D), 0)
    upd = jnp.where(rows == 0, s1, 0.0) + jnp.where(rows == 1, s2, 0.0)

    @pl.when(i == 0)
    def _():
        st_ref[...] = jnp.zeros_like(st_ref)

    st_ref[...] += upd


def _layer1_body(x_ref, af_ref, e0_ref, e1_ref, b1_ref, w1_ref, w2_ref,
                 b2_ref, h_ref, st_ref):
    i = pl.program_id(0)
    a = jnp.concatenate([af_ref[...], e0_ref[...] + e1_ref[...]], axis=1)
    z = x_ref[...] + a
    z = jnp.concatenate([z, jnp.zeros((BLK, 128 - F1), jnp.float32)],
                        axis=1)
    zr = jnp.maximum(_bdot(z, w1_ref[...]) + b1_ref[0:1, :], 0.0)
    h = _bdot(zr, w2_ref[...]) + b2_ref[0:1, :]
    h = jnp.where(h > 0, h, jnp.exp(jnp.minimum(h, 0.0)) - 1.0)
    h_ref[...] = h
    _stats_update(st_ref, h, i)


def _layer1(x_pad, aggf, e0, e1, b1r, w1_pad, w2, b2r):
    return pl.pallas_call(
        _layer1_body,
        grid=(NB,),
        in_specs=[
            pl.BlockSpec((BLK, F1), lambda i: (i, 0)),
            pl.BlockSpec((BLK, 2 * FH), lambda i: (i, 0)),
            pl.BlockSpec((BLK, FT), lambda i: (i, 0)),
            pl.BlockSpec((BLK, FT), lambda i: (i, 0)),
            pl.BlockSpec((8, D), lambda i: (0, 0)),
            pl.BlockSpec((128, D), lambda i: (0, 0)),
            pl.BlockSpec((D, D), lambda i: (0, 0)),
            pl.BlockSpec((8, D), lambda i: (0, 0)),
        ],
        out_specs=[
            pl.BlockSpec((BLK, D), lambda i: (i, 0)),
            pl.BlockSpec((8, D), lambda i: (0, 0)),
        ],
        out_shape=[
            jax.ShapeDtypeStruct((N, D), jnp.float32),
            jax.ShapeDtypeStruct((8, D), jnp.float32),
        ],
    )(x_pad, aggf, e0, e1, b1r, w1_pad, w2, b2r)


def _bn_body(h_ref, st_ref, o_ref):
    mu = st_ref[0:1, :] * (1.0 / N)
    ex2 = st_ref[1:2, :] * (1.0 / N)
    inv = lax.rsqrt(ex2 - mu * mu + 1e-5)
    o_ref[...] = (h_ref[...] - mu) * inv


def _bn_apply(h, stats):
    return pl.pallas_call(
        _bn_body,
        grid=(NB,),
        in_specs=[
            pl.BlockSpec((BLK, D), lambda i: (i, 0)),
            pl.BlockSpec((8, D), lambda i: (0, 0)),
        ],
        out_specs=pl.BlockSpec((BLK, D), lambda i: (i, 0)),
        out_shape=jax.ShapeDtypeStruct((N, D), jnp.float32),
    )(h, stats)


def _layer_body(pool, hn_ref, a0_ref, a1_ref, b1_ref, w1_ref, w2_ref,
                b2_ref, batch_ref, h_ref, st_ref, pooled_ref):
    i = pl.program_id(0)
    z = hn_ref[...] + a0_ref[...] + a1_ref[...]
    zr = jnp.maximum(_bdot(z, w1_ref[...]) + b1_ref[0:1, :], 0.0)
    h = _bdot(zr, w2_ref[...]) + b2_ref[0:1, :]
    h = jnp.maximum(h, 0.0)
    h_ref[...] = h
    _stats_update(st_ref, h, i)

    if pool:
        bb = batch_ref[0, 0, :].reshape(1, BLK)
        gio = lax.broadcasted_iota(jnp.int32, (G, BLK), 0)
        oht = (gio == bb).astype(jnp.float32)
        hx = jnp.concatenate(
            [h, jnp.ones((BLK, 8), jnp.float32)], axis=1)
        pu = jnp.dot(oht, hx, preferred_element_type=jnp.float32,
                     precision=lax.Precision.HIGHEST)

        @pl.when(i == 0)
        def _():
            pooled_ref[...] = jnp.zeros_like(pooled_ref)

        pooled_ref[...] += pu


def _layer(pool, hn, a0, a1, b1r, w1, w2, b2r, batch3d):
    body = functools.partial(_layer_body, pool)
    return pl.pallas_call(
        body,
        grid=(NB,),
        in_specs=[
            pl.BlockSpec((BLK, D), lambda i: (i, 0)),
            pl.BlockSpec((BLK, D), lambda i: (i, 0)),
            pl.BlockSpec((BLK, D), lambda i: (i, 0)),
            pl.BlockSpec((8, D), lambda i: (0, 0)),
            pl.BlockSpec((D, D), lambda i: (0, 0)),
            pl.BlockSpec((D, D), lambda i: (0, 0)),
            pl.BlockSpec((8, D), lambda i: (0, 0)),
            pl.BlockSpec((1, 1, BLK), lambda i: (i, 0, 0)),
        ],
        out_specs=[
            pl.BlockSpec((BLK, D), lambda i: (i, 0)),
            pl.BlockSpec((8, D), lambda i: (0, 0)),
            pl.BlockSpec((G, D + 8), lambda i: (0, 0)),
        ],
        out_shape=[
            jax.ShapeDtypeStruct((N, D), jnp.float32),
            jax.ShapeDtypeStruct((8, D), jnp.float32),
            jax.ShapeDtypeStruct((G, D + 8), jnp.float32),
        ],
    )(hn, a0, a1, b1r, w1, w2, b2r, batch3d)


def _head_body(pooled_ref, st_ref, target_ref, g_ref, bb_ref, n2w_ref,
               n2b_ref, n31w_ref, n31b_ref, n32w_ref, n32b_ref, n4t_ref,
               n4b_ref, n4bias_ref, n5w_ref, n5b_ref, out_ref, xg_ref):
    mu = st_ref[0:1, :] * (1.0 / N)
    ex2 = st_ref[1:2, :] * (1.0 / N)
    inv = lax.rsqrt(ex2 - mu * mu + 1e-5)
    praw = pooled_ref[:, 0:D]
    cnt = pooled_ref[:, D:D + 1]
    pooled = (praw - cnt * mu) * inv

    xg = _bdot(pooled, n2w_ref[...])
    xg = jnp.maximum(xg + n2b_ref[0:1, :], 0.0)
    xg_ref[...] = xg

    t = target_ref[...]
    tm = jnp.mean(t, axis=0, keepdims=True)
    tv = jnp.mean((t - tm) ** 2, axis=0, keepdims=True)
    tn = (t - tm) / jnp.sqrt(tv + 1e-5)
    tn = tn * g_ref[0:1, :] + bb_ref[0:1, :]

    c = _bdot(tn, n31w_ref[...]) + n31b_ref[0:1, :]
    c = _bdot(c, n32w_ref[...]) + n32b_ref[0:1, :]
    c = c - jnp.max(c, axis=1, keepdims=True)
    ec = jnp.exp(c)
    sm = ec / jnp.sum(ec, axis=1, keepdims=True)

    xc = (_bdot(xg, n4t_ref[...]) + _bdot(sm, n4b_ref[...])
          + n4bias_ref[0:1, :])
    xc = jnp.maximum(xc, 0.0)

    o = _bdot(xc, n5w_ref[...])
    o = o[:, 0:1] + n5b_ref[0:1, 0:1]
    out_ref[...] = jax.nn.sigmoid(o)


def _head(pooled, st3, target, g8, b8, n2w, n2b8, n31w, n31b8, n32w, n32b8,
          n4t, n4b, n4bias8, n5wp, n5b8):
    return pl.pallas_call(
        _head_body,
        out_shape=[
            jax.ShapeDtypeStruct((G, 1), jnp.float32),
            jax.ShapeDtypeStruct((G, 128), jnp.float32),
        ],
    )(pooled, st3, target, g8, b8, n2w, n2b8, n31w, n31b8, n32w, n32b8,
      n4t, n4b, n4bias8, n5wp, n5b8)


def _row8(v):
    return jnp.broadcast_to(v.reshape(1, -1), (8, v.shape[0]))


def _pad_edges(v, ntiles, tpt, fill):
    q = E // ntiles
    p = tpt * BSZ - q
    vt = v.reshape(ntiles, q)
    if fill == 0:
        pad = jnp.zeros((ntiles, p), jnp.int32)
    else:
        pad = jnp.broadcast_to(
            N + (jnp.arange(p, dtype=jnp.int32) % (NPAD - N)), (ntiles, p))
    return jnp.concatenate([vt, pad], axis=1).reshape(-1, BSZ)


def kernel(x, edge_index, batch, target, n11_W1, n11_b1, n11_W2, n11_b2,
           n12_W1, n12_b1, n12_W2, n12_b2, n13_W1, n13_b1, n13_W2, n13_b2,
           n2_W, n2_b, n31_W, n31_b, n32_W, n32_b, n4_W, n4_b, n5_W, n5_b,
           bn1_g, bn1_b):
    src = edge_index[0]
    dst = edge_index[1]
    src2d_w = _pad_edges(src, NW, BPT_W, 0)
    dst2d_w = _pad_edges(dst, NW, BPT_W, N)
    src2d_f = _pad_edges(src, NTILE, BPT_F, 0)
    dst2d_f = _pad_edges(dst, NTILE, BPT_F, N)
    zeros_d = jnp.zeros((RPT, D), jnp.float32)
    zeros_f = jnp.zeros((RPT, FH), jnp.float32)
    zeros_t = jnp.zeros((RPT, FT), jnp.float32)
    x_pad = jnp.pad(x, ((0, 0), (0, F1 - x.shape[1])))
    w1_pad = jnp.pad(n11_W1, ((0, 128 - n11_W1.shape[0]), (0, 0)))
    batch3d = batch.reshape(NB, 1, BLK)

    aggf = _sc_aggregate_x(x_pad[:, :FH], x_pad[:, FH:2 * FH],
                           src2d_f, dst2d_f, zeros_f)
    aggf = jnp.concatenate([aggf[0, :N], aggf[1, :N]], axis=1)
    aggt = _sc_aggregate(x_pad[:, 2 * FH:], src2d_w, dst2d_w, zeros_t,
                         width=FT)
    h1, st1 = _layer1(x_pad, aggf, aggt[0, :N], aggt[1, :N],
                      _row8(n11_b1), w1_pad, n11_W2, _row8(n11_b2))

    h1n = _bn_apply(h1, st1)
    agg2 = _sc_aggregate(h1n, src2d_w, dst2d_w, zeros_d)
    h2, st2, _ = _layer(False, h1n, agg2[0, :N], agg2[1, :N],
                        _row8(n12_b1), n12_W1, n12_W2, _row8(n12_b2),
                        batch3d)

    h2n = _bn_apply(h2, st2)
    agg3 = _sc_aggregate(h2n, src2d_w, dst2d_w, zeros_d)
    _, st3, pooled = _layer(True, h2n, agg3[0, :N], agg3[1, :N],
                            _row8(n13_b1), n13_W1, n13_W2, _row8(n13_b2),
                            batch3d)

    n4t, n4b = n4_W[:128], n4_W[128:]
    n5wp = jnp.pad(n5_W, ((0, 0), (0, 7)))
    out, xg = _head(pooled, st3, target, _row8(bn1_g), _row8(bn1_b),
                    n2_W, _row8(n2_b), n31_W, _row8(n31_b), n32_W,
                    _row8(n32_b), n4t, n4b, _row8(n4_b), n5wp, _row8(n5_b))
    return (out, xg)


# speedup vs baseline: 6.9627x; 1.0000x over previous
import functools

import jax
import jax.numpy as jnp
from jax import lax
from jax.experimental import pallas as pl
from jax.experimental.pallas import tpu as pltpu
from jax.experimental.pallas import tpu_sc as plsc

N = 50000
E = 800000
G = 512
BLK = 2000
NB = N // BLK
D = 32
F1 = 80
FH = 32
FT = 16

NSC = 2
NTILE = 16
NW = NSC * NTILE
CH = 128
K = 3
BSZ = K * CH
NPAD = 50048
RPT = NPAD // NTILE

BPT_W = 66
EPAD_W = NW * BPT_W * BSZ
BPT_F = 131
EPAD_F = NTILE * BPT_F * BSZ

BF = jnp.bfloat16


def _bdot(a, b):
    return jnp.dot(a.astype(BF), b.astype(BF),
                   preferred_element_type=jnp.float32)


def _edge_loop(table_hbm, src_hbm, dst_hbm, acc_sh, src_v, dst_v, rows_v,
               gsems, ssems, isems, base, tpt):
    blocks = tpt
    g = {}
    s = {}
    i = {}
    pltpu.sync_copy(src_hbm.at[base], src_v.at[0])
    pltpu.sync_copy(dst_hbm.at[base], dst_v.at[0])
    for b in range(blocks):
        b2 = b % 2
        b3 = b % 3
        if b >= 2:
            s[b - 2].wait()
        if b >= 1:
            for d in i[b]:
                d.wait()
        g[b] = pltpu.async_copy(table_hbm.at[src_v.at[b3]], rows_v.at[b2],
                                gsems[b2])
        if b + 1 < blocks:
            n3 = (b + 1) % 3
            i[b + 1] = (
                pltpu.async_copy(src_hbm.at[base + b + 1], src_v.at[n3],
                                 isems[(b + 1) % 2]),
                pltpu.async_copy(dst_hbm.at[base + b + 1], dst_v.at[n3],
                                 isems[(b + 1) % 2]),
            )
        g[b].wait()
        s[b] = pltpu.async_copy(rows_v.at[b2], acc_sh.at[dst_v.at[b3]],
                                ssems[b2], add=True)
    if blocks >= 2:
        s[blocks - 2].wait()
    s[blocks - 1].wait()


def _sc_agg_edges_body(p_hbm, src_hbm, dst_hbm, zeros_hbm, out_hbm,
                       src_v, dst_v, rows_v, acc_sh, gsems, ssems, isems):
    c = lax.axis_index("c")
    s = lax.axis_index("s")
    wid = c * NTILE + s
    pltpu.sync_copy(zeros_hbm, acc_sh.at[pl.ds(s * RPT, RPT)])
    plsc.subcore_barrier()
    _edge_loop(p_hbm, src_hbm, dst_hbm, acc_sh, src_v, dst_v, rows_v,
               gsems, ssems, isems, wid * BPT_W, BPT_W)
    plsc.subcore_barrier()
    pltpu.sync_copy(acc_sh.at[pl.ds(s * RPT, RPT)],
                    out_hbm.at[c, pl.ds(s * RPT, RPT)])


def _sc_aggregate(p, src2d, dst2d, zeros_init, width=D):
    mesh = plsc.VectorSubcoreMesh(core_axis_name="c", subcore_axis_name="s")
    f = pl.kernel(
        _sc_agg_edges_body,
        out_type=jax.ShapeDtypeStruct((NSC, NPAD, width), jnp.float32),
        mesh=mesh,
        scratch_types=[
            pltpu.VMEM((3, BSZ), jnp.int32),
            pltpu.VMEM((3, BSZ), jnp.int32),
            pltpu.VMEM((2, BSZ, width), jnp.float32),
            pltpu.VMEM_SHARED((NPAD, width), jnp.float32),
            [pltpu.SemaphoreType.DMA] * 2,
            [pltpu.SemaphoreType.DMA] * 2,
            [pltpu.SemaphoreType.DMA] * 2,
        ],
        compiler_params=pltpu.CompilerParams(use_tc_tiling_on_sc=False),
        name="gin_edge_agg",
    )
    return f(p, src2d, dst2d, zeros_init)


def _sc_agg_feat_body(xl_hbm, xr_hbm, src_hbm, dst_hbm, zeros_hbm, out_hbm,
                      src_v, dst_v, rows_v, acc_sh, gsems, ssems, isems):
    c = lax.axis_index("c")
    s = lax.axis_index("s")
    pltpu.sync_copy(zeros_hbm, acc_sh.at[pl.ds(s * RPT, RPT)])
    plsc.subcore_barrier()

    @pl.when(c == 0)
    def _():
        _edge_loop(xl_hbm, src_hbm, dst_hbm, acc_sh, src_v, dst_v, rows_v,
                   gsems, ssems, isems, s * BPT_F, BPT_F)

    @pl.when(c == 1)
    def _():
        _edge_loop(xr_hbm, src_hbm, dst_hbm, acc_sh, src_v, dst_v, rows_v,
                   gsems, ssems, isems, s * BPT_F, BPT_F)

    plsc.subcore_barrier()
    pltpu.sync_copy(acc_sh.at[pl.ds(s * RPT, RPT)],
                    out_hbm.at[c, pl.ds(s * RPT, RPT)])


def _sc_aggregate_x(xl, xr, src2d, dst2d, zeros_init):
    mesh = plsc.VectorSubcoreMesh(core_axis_name="c", subcore_axis_name="s")
    f = pl.kernel(
        _sc_agg_feat_body,
        out_type=jax.ShapeDtypeStruct((NSC, NPAD, FH), jnp.float32),
        mesh=mesh,
        scratch_types=[
            pltpu.VMEM((3, BSZ), jnp.int32),
            pltpu.VMEM((3, BSZ), jnp.int32),
            pltpu.VMEM((2, BSZ, FH), jnp.float32),
            pltpu.VMEM_SHARED((NPAD, FH), jnp.float32),
            [pltpu.SemaphoreType.DMA] * 2,
            [pltpu.SemaphoreType.DMA] * 2,
            [pltpu.SemaphoreType.DMA] * 2,
        ],
        compiler_params=pltpu.CompilerParams(use_tc_tiling_on_sc=False),
        name="gin_x_agg",
    )
    return f(xl, xr, src2d, dst2d, zeros_init)


def _stats_update(st_ref, h, i):
    s1 = jnp.sum(h, axis=0, keepdims=True)
    s2 = jnp.sum(h * h, axis=0, keepdims=True)
    rows = lax.broadcasted_iota(jnp.int32, (8, D), 0)
    upd = jnp.where(rows == 0, s1, 0.0) + jnp.where(rows == 1, s2, 0.0)

    @pl.when(i == 0)
    def _():
        st_ref[...] = jnp.zeros_like(st_ref)

    st_ref[...] += upd


def _layer1_body(x_ref, af_ref, e0_ref, e1_ref, b1_ref, w1_ref, w2_ref,
                 b2_ref, h_ref, st_ref):
    i = pl.program_id(0)
    a = jnp.concatenate([af_ref[...], e0_ref[...] + e1_ref[...]], axis=1)
    z = x_ref[...] + a
    z = jnp.concatenate([z, jnp.zeros((BLK, 128 - F1), jnp.float32)],
                        axis=1)
    zr = jnp.maximum(_bdot(z, w1_ref[...]) + b1_ref[0:1, :], 0.0)
    h = _bdot(zr, w2_ref[...]) + b2_ref[0:1, :]
    h = jnp.where(h > 0, h, jnp.exp(jnp.minimum(h, 0.0)) - 1.0)
    h_ref[...] = h
    _stats_update(st_ref, h, i)


def _layer1(x_pad, aggf, e0, e1, b1r, w1_pad, w2, b2r):
    return pl.pallas_call(
        _layer1_body,
        grid=(NB,),
        in_specs=[
            pl.BlockSpec((BLK, F1), lambda i: (i, 0)),
            pl.BlockSpec((BLK, 2 * FH), lambda i: (i, 0)),
            pl.BlockSpec((BLK, FT), lambda i: (i, 0)),
            pl.BlockSpec((BLK, FT), lambda i: (i, 0)),
            pl.BlockSpec((8, D), lambda i: (0, 0)),
            pl.BlockSpec((128, D), lambda i: (0, 0)),
            pl.BlockSpec((D, D), lambda i: (0, 0)),
            pl.BlockSpec((8, D), lambda i: (0, 0)),
        ],
        out_specs=[
            pl.BlockSpec((BLK, D), lambda i: (i, 0)),
            pl.BlockSpec((8, D), lambda i: (0, 0)),
        ],
        out_shape=[
            jax.ShapeDtypeStruct((N, D), jnp.float32),
            jax.ShapeDtypeStruct((8, D), jnp.float32),
        ],
    )(x_pad, aggf, e0, e1, b1r, w1_pad, w2, b2r)


def _bn_body(h_ref, st_ref, o_ref):
    mu = st_ref[0:1, :] * (1.0 / N)
    ex2 = st_ref[1:2, :] * (1.0 / N)
    inv = lax.rsqrt(ex2 - mu * mu + 1e-5)
    o_ref[...] = (h_ref[...] - mu) * inv


def _bn_apply(h, stats):
    return pl.pallas_call(
        _bn_body,
        grid=(NB,),
        in_specs=[
            pl.BlockSpec((BLK, D), lambda i: (i, 0)),
            pl.BlockSpec((8, D), lambda i: (0, 0)),
        ],
        out_specs=pl.BlockSpec((BLK, D), lambda i: (i, 0)),
        out_shape=jax.ShapeDtypeStruct((N, D), jnp.float32),
    )(h, stats)


def _layer_body(pool, hn_ref, a0_ref, a1_ref, b1_ref, w1_ref, w2_ref,
                b2_ref, batch_ref, h_ref, st_ref, pooled_ref):
    i = pl.program_id(0)
    z = hn_ref[...] + a0_ref[...] + a1_ref[...]
    zr = jnp.maximum(_bdot(z, w1_ref[...]) + b1_ref[0:1, :], 0.0)
    h = _bdot(zr, w2_ref[...]) + b2_ref[0:1, :]
    h = jnp.maximum(h, 0.0)
    h_ref[...] = h
    _stats_update(st_ref, h, i)

    if pool:
        bb = batch_ref[0, 0, :].reshape(1, BLK)
        gio = lax.broadcasted_iota(jnp.int32, (G, BLK), 0)
        oht = (gio == bb).astype(jnp.float32)
        hx = jnp.concatenate(
            [h, jnp.ones((BLK, 8), jnp.float32)], axis=1)
        pu = jnp.dot(oht, hx, preferred_element_type=jnp.float32,
                     precision=lax.Precision.HIGHEST)

        @pl.when(i == 0)
        def _():
            pooled_ref[...] = jnp.zeros_like(pooled_ref)

        pooled_ref[...] += pu


def _layer(pool, hn, a0, a1, b1r, w1, w2, b2r, batch3d):
    body = functools.partial(_layer_body, pool)
    return pl.pallas_call(
        body,
        grid=(NB,),
        in_specs=[
            pl.BlockSpec((BLK, D), lambda i: (i, 0)),
            pl.BlockSpec((BLK, D), lambda i: (i, 0)),
            pl.BlockSpec((BLK, D), lambda i: (i, 0)),
            pl.BlockSpec((8, D), lambda i: (0, 0)),
            pl.BlockSpec((D, D), lambda i: (0, 0)),
            pl.BlockSpec((D, D), lambda i: (0, 0)),
            pl.BlockSpec((8, D), lambda i: (0, 0)),
            pl.BlockSpec((1, 1, BLK), lambda i: (i, 0, 0)),
        ],
        out_specs=[
            pl.BlockSpec((BLK, D), lambda i: (i, 0)),
            pl.BlockSpec((8, D), lambda i: (0, 0)),
            pl.BlockSpec((G, D + 8), lambda i: (0, 0)),
        ],
        out_shape=[
            jax.ShapeDtypeStruct((N, D), jnp.float32),
            jax.ShapeDtypeStruct((8, D), jnp.float32),
            jax.ShapeDtypeStruct((G, D + 8), jnp.float32),
        ],
    )(hn, a0, a1, b1r, w1, w2, b2r, batch3d)


def _head_body(pooled_ref, st_ref, target_ref, g_ref, bb_ref, n2w_ref,
               n2b_ref, n31w_ref, n31b_ref, n32w_ref, n32b_ref, n4t_ref,
               n4b_ref, n4bias_ref, n5w_ref, n5b_ref, out_ref, xg_ref):
    mu = st_ref[0:1, :] * (1.0 / N)
    ex2 = st_ref[1:2, :] * (1.0 / N)
    inv = lax.rsqrt(ex2 - mu * mu + 1e-5)
    praw = pooled_ref[:, 0:D]
    cnt = pooled_ref[:, D:D + 1]
    pooled = (praw - cnt * mu) * inv

    xg = _bdot(pooled, n2w_ref[...])
    xg = jnp.maximum(xg + n2b_ref[0:1, :], 0.0)
    xg_ref[...] = xg

    t = target_ref[...]
    tm = jnp.mean(t, axis=0, keepdims=True)
    tv = jnp.mean((t - tm) ** 2, axis=0, keepdims=True)
    tn = (t - tm) / jnp.sqrt(tv + 1e-5)
    tn = tn * g_ref[0:1, :] + bb_ref[0:1, :]

    c = _bdot(tn, n31w_ref[...]) + n31b_ref[0:1, :]
    c = _bdot(c, n32w_ref[...]) + n32b_ref[0:1, :]
    c = c - jnp.max(c, axis=1, keepdims=True)
    ec = jnp.exp(c)
    sm = ec / jnp.sum(ec, axis=1, keepdims=True)

    xc = (_bdot(xg, n4t_ref[...]) + _bdot(sm, n4b_ref[...])
          + n4bias_ref[0:1, :])
    xc = jnp.maximum(xc, 0.0)

    o = _bdot(xc, n5w_ref[...])
    o = o[:, 0:1] + n5b_ref[0:1, 0:1]
    out_ref[...] = jax.nn.sigmoid(o)


def _head(pooled, st3, target, g8, b8, n2w, n2b8, n31w, n31b8, n32w, n32b8,
          n4t, n4b, n4bias8, n5wp, n5b8):
    return pl.pallas_call(
        _head_body,
        out_shape=[
            jax.ShapeDtypeStruct((G, 1), jnp.float32),
            jax.ShapeDtypeStruct((G, 128), jnp.float32),
        ],
    )(pooled, st3, target, g8, b8, n2w, n2b8, n31w, n31b8, n32w, n32b8,
      n4t, n4b, n4bias8, n5wp, n5b8)


def _row8(v):
    return jnp.broadcast_to(v.reshape(1, -1), (8, v.shape[0]))


def _pad_edges(v, ntiles, tpt, fill):
    q = E // ntiles
    p = tpt * BSZ - q
    vt = v.reshape(ntiles, q)
    if fill == 0:
        pad = jnp.zeros((ntiles, p), jnp.int32)
    else:
        pad = jnp.broadcast_to(
            N + (jnp.arange(p, dtype=jnp.int32) % (NPAD - N)), (ntiles, p))
    return jnp.concatenate([vt, pad], axis=1).reshape(-1, BSZ)


def kernel(x, edge_index, batch, target, n11_W1, n11_b1, n11_W2, n11_b2,
           n12_W1, n12_b1, n12_W2, n12_b2, n13_W1, n13_b1, n13_W2, n13_b2,
           n2_W, n2_b, n31_W, n31_b, n32_W, n32_b, n4_W, n4_b, n5_W, n5_b,
           bn1_g, bn1_b):
    src = edge_index[0]
    dst = edge_index[1]
    src2d_w = _pad_edges(src, NW, BPT_W, 0)
    dst2d_w = _pad_edges(dst, NW, BPT_W, N)
    src2d_f = _pad_edges(src, NTILE, BPT_F, 0)
    dst2d_f = _pad_edges(dst, NTILE, BPT_F, N)
    zeros_d = jnp.zeros((RPT, D), jnp.float32)
    zeros_f = jnp.zeros((RPT, FH), jnp.float32)
    zeros_t = jnp.zeros((RPT, FT), jnp.float32)
    x_pad = jnp.pad(x, ((0, 0), (0, F1 - x.shape[1])))
    w1_pad = jnp.pad(n11_W1, ((0, 128 - n11_W1.shape[0]), (0, 0)))
    batch3d = batch.reshape(NB, 1, BLK)

    aggf = _sc_aggregate_x(x_pad[:, :FH], x_pad[:, FH:2 * FH],
                           src2d_f, dst2d_f, zeros_f)
    aggf = jnp.concatenate([aggf[0, :N], aggf[1, :N]], axis=1)
    aggt = _sc_aggregate(x_pad[:, 2 * FH:], src2d_w, dst2d_w, zeros_t,
                         width=FT)
    h1, st1 = _layer1(x_pad, aggf, aggt[0, :N], aggt[1, :N],
                      _row8(n11_b1), w1_pad, n11_W2, _row8(n11_b2))

    h1n = _bn_apply(h1, st1)
    agg2 = _sc_aggregate(h1n, src2d_w, dst2d_w, zeros_d)
    h2, st2, _ = _layer(False, h1n, agg2[0, :N], agg2[1, :N],
                        _row8(n12_b1), n12_W1, n12_W2, _row8(n12_b2),
                        batch3d)

    h2n = _bn_apply(h2, st2)
    agg3 = _sc_aggregate(h2n, src2d_w, dst2d_w, zeros_d)
    _, st3, pooled = _layer(True, h2n, agg3[0, :N], agg3[1, :N],
                            _row8(n13_b1), n13_W1, n13_W2, _row8(n13_b2),
                            batch3d)

    n4t, n4b = n4_W[:128], n4_W[128:]
    n5wp = jnp.pad(n5_W, ((0, 0), (0, 7)))
    out, xg = _head(pooled, st3, target, _row8(bn1_g), _row8(bn1_b),
                    n2_W, _row8(n2_b), n31_W, _row8(n31_b), n32_W,
                    _row8(n32_b), n4t, n4b, _row8(n4_b), n5wp, _row8(n5_b))
    return (out, xg)
